# Initial kernel scaffold; baseline (speedup 1.0000x reference)
import jax
import jax.numpy as jnp
from jax.experimental import pallas as pl


def kernel(x, edge_index, datanorm, W_in1, b_in1, W_in2, b_in2, W_in3, b_in3, W_c0a, b_c0a, W_c0b, b_c0b, W_c1a, b_c1a, W_c1b, b_c1b, W_e1, b_e1, W_e2, b_e2, W_e3, b_e3):
    raise NotImplementedError("write your pallas kernel here")


# speedup vs baseline: 1.7915x; 1.7915x over previous
import functools

import jax
import jax.numpy as jnp
from jax import lax
from jax.experimental import pallas as pl
from jax.experimental.pallas import tpu as pltpu
from jax.experimental.pallas import tpu_sc as plsc

f32 = jnp.float32
PREC = lax.Precision.HIGHEST

NP = 10240
MWP = 256
BN = 2048
BE = 2560
NC, NS = 2, 16
NWK = NC * NS
CG = 80
CS = 200


def _elu(v):
    return jnp.where(v > 0, v, jnp.exp(jnp.minimum(v, 0.0)) - 1.0)


def _dot(a, b):
    return jax.lax.dot_general(a, b, (((1,), (0,)), ((), ())),
                               precision=PREC, preferred_element_type=f32)


def _node0_body(xr, dnr, w1r, b1r, w2r, b2r, w3r, b3r,
                ahr, axr, bhr, bxr, bar, p_out, q_out):
    xn = xr[...] * dnr[...]
    h = jnp.tanh(_dot(xn, w1r[...]) + b1r[...])
    h = jnp.tanh(_dot(h, w2r[...]) + b2r[...])
    h0 = jnp.tanh(_dot(h, w3r[...]) + b3r[...])
    p_out[...] = _dot(h0, ahr[...]) + _dot(xn, axr[...]) + bar[...]
    q_out[...] = _dot(h0, bhr[...]) + _dot(xn, bxr[...])


def _node1_body(hpr, xr, dnr, ahr, axr, bhr, bxr, bar,
                h1_out, p_out, q_out):
    h1 = hpr[0] + hpr[1]
    xn = xr[...] * dnr[...]
    h1_out[...] = h1
    p_out[...] = _dot(h1, ahr[...]) + _dot(xn, axr[...]) + bar[...]
    q_out[...] = _dot(h1, bhr[...]) + _dot(xn, bxr[...])


def _node2_body(hpr, h1r, wr2, wr1, ws2, ws1, ber, r_out, s_out):
    h2 = hpr[0] + hpr[1]
    h1 = h1r[...]
    r_out[...] = _dot(h2, wr2[...]) + _dot(h1, wr1[...]) + ber[...]
    s_out[...] = _dot(h2, ws2[...]) + _dot(h1, ws1[...])


def _econv_body(pr, qr, wbr, bbr, m_out):
    u = _elu(pr[...] + qr[...])
    m_out[...] = _elu(_dot(u, wbr[...]) + bbr[...])


def _efin_body(rr, sr, w2r, b2r, w3r, b3r, o_out):
    e1 = _elu(rr[...] + sr[...])
    e2 = _elu(_dot(e1, w2r[...]) + b2r[...])
    lg = _dot(e2, w3r[...]) + b3r[...]
    mx = jnp.max(lg, axis=-1, keepdims=True)
    sh = lg - mx
    lse = jnp.log(jnp.sum(jnp.exp(sh), axis=-1, keepdims=True))
    o_out[...] = sh - lse


def _full(shape):
    return pl.BlockSpec(shape, lambda i: (0,) * len(shape))


def _blk(shape):
    def im(i):
        return (i,) + (0,) * (len(shape) - 1)
    return pl.BlockSpec(shape, im)


def _blk2(shape):
    def im(i):
        return (0, i) + (0,) * (len(shape) - 2)
    return pl.BlockSpec(shape, im)


def _make_gather2(E, W):
    perw = E // NWK
    nch = perw // CG
    mesh = plsc.VectorSubcoreMesh(core_axis_name="c", subcore_axis_name="s")

    @functools.partial(
        pl.kernel, mesh=mesh,
        out_type=[jax.ShapeDtypeStruct((E, W), f32),
                  jax.ShapeDtypeStruct((E, W), f32)],
        scratch_types=[pltpu.VMEM((CG,), jnp.int32),
                       pltpu.VMEM((CG,), jnp.int32),
                       pltpu.VMEM((CG, W), f32),
                       pltpu.VMEM((CG, W), f32),
                       pltpu.SemaphoreType.DMA,
                       pltpu.SemaphoreType.DMA],
    )
    def gather2(ta, tb, ia, ib, oa, ob, iva, ivb, ra, rb, sema, semb):
        wid = lax.axis_index("s") * NC + lax.axis_index("c")
        base = wid * perw

        def body(i, carry):
            off = pl.multiple_of(base + i * CG, 8)
            pltpu.sync_copy(ia.at[pl.ds(off, CG)], iva)
            pltpu.sync_copy(ib.at[pl.ds(off, CG)], ivb)
            cpa = pltpu.async_copy(ta.at[iva], ra, sema)
            cpb = pltpu.async_copy(tb.at[ivb], rb, semb)
            cpa.wait()
            cpb.wait()
            pltpu.sync_copy(ra, oa.at[pl.ds(off, CG)])
            pltpu.sync_copy(rb, ob.at[pl.ds(off, CG)])
            return carry

        lax.fori_loop(0, nch, body, 0)

    return gather2


def _make_scatter(E, H):
    perw = E // NWK
    nch = perw // CS
    rows_per_tile = NP // NS
    mesh = plsc.VectorSubcoreMesh(core_axis_name="c", subcore_axis_name="s")

    @functools.partial(
        pl.kernel, mesh=mesh,
        out_type=jax.ShapeDtypeStruct((NC, NP, H), f32),
        scratch_types=[pltpu.VMEM((CS,), jnp.int32),
                       pltpu.VMEM((CS, H), f32),
                       pltpu.VMEM_SHARED((NP, H), f32)],
    )
    def scat(mref, cref, zref, out, iv, rv, acc):
        c = lax.axis_index("c")
        s = lax.axis_index("s")
        r0 = s * rows_per_tile
        pltpu.sync_copy(zref.at[pl.ds(r0, rows_per_tile)],
                        acc.at[pl.ds(r0, rows_per_tile)])
        plsc.subcore_barrier()
        base = (s * NC + c) * perw

        def body(i, carry):
            off = pl.multiple_of(base + i * CS, 8)
            pltpu.sync_copy(cref.at[pl.ds(off, CS)], iv)
            pltpu.sync_copy(mref.at[pl.ds(off, CS)], rv)
            pltpu.sync_copy(rv, acc.at[iv], add=True)
            return carry

        lax.fori_loop(0, nch, body, 0)
        plsc.subcore_barrier()
        pltpu.sync_copy(acc.at[pl.ds(r0, rows_per_tile)],
                        out.at[c, pl.ds(r0, rows_per_tile)])

    return scat


def kernel(x, edge_index, datanorm, W_in1, b_in1, W_in2, b_in2, W_in3,
           b_in3, W_c0a, b_c0a, W_c0b, b_c0b, W_c1a, b_c1a, W_c1b, b_c1b,
           W_e1, b_e1, W_e2, b_e2, W_e3, b_e3):
    N, D = x.shape
    E = edge_index.shape[1]
    H = W_in3.shape[1]
    F = H + D

    row = edge_index[0]
    col = edge_index[1]

    def pad2(w, r, c):
        return jnp.zeros((r, c), f32).at[:w.shape[0], :w.shape[1]].set(w)

    xp = jnp.zeros((NP, 8), f32).at[:N, :D].set(x)
    dn = jnp.zeros((1, 8), f32).at[0, :D].set(datanorm)
    zeros_nh = jnp.zeros((NP, H), f32)

    W1p = pad2(W_in1, 8, 2 * H)
    b1p = b_in1[None]
    b2p = b_in2[None]
    b3p = b_in3[None]

    def conv_prep(Wa, ba, Wb, bb):
        A = Wa[:F] - Wa[F:]
        B = Wa[F:]
        return (pad2(A[:H], H, MWP), pad2(A[H:], 8, MWP),
                pad2(B[:H], H, MWP), pad2(B[H:], 8, MWP),
                pad2(ba[None], 1, MWP), pad2(Wb, MWP, H), bb[None])

    A0h, A0x, B0h, B0x, ba0, Wb0, bb0 = conv_prep(W_c0a, b_c0a, W_c0b, b_c0b)
    A1h, A1x, B1h, B1x, ba1, Wb1, bb1 = conv_prep(W_c1a, b_c1a, W_c1b, b_c1b)

    Wr2, Wr1 = W_e1[:H], W_e1[H:2 * H]
    Ws2, Ws1 = W_e1[2 * H:3 * H], W_e1[3 * H:]
    be1 = b_e1[None]
    be2 = b_e2[None]
    W3p = pad2(W_e3, 2 * H, 8)
    b3f = jnp.full((1, 8), -1e30, f32).at[0, :W_e3.shape[1]].set(b_e3)

    gn = NP // BN
    ge = E // BE

    node0 = pl.pallas_call(
        _node0_body,
        grid=(gn,),
        in_specs=[_blk((BN, 8)), _full((1, 8)),
                  _full((8, 2 * H)), _full((1, 2 * H)),
                  _full((2 * H, 2 * H)), _full((1, 2 * H)),
                  _full((2 * H, H)), _full((1, H)),
                  _full((H, MWP)), _full((8, MWP)),
                  _full((H, MWP)), _full((8, MWP)), _full((1, MWP))],
        out_specs=[_blk((BN, MWP)), _blk((BN, MWP))],
        out_shape=[jax.ShapeDtypeStruct((NP, MWP), f32)] * 2,
    )

    node1 = pl.pallas_call(
        _node1_body,
        grid=(gn,),
        in_specs=[_blk2((NC, BN, H)), _blk((BN, 8)), _full((1, 8)),
                  _full((H, MWP)), _full((8, MWP)),
                  _full((H, MWP)), _full((8, MWP)), _full((1, MWP))],
        out_specs=[_blk((BN, H)), _blk((BN, MWP)), _blk((BN, MWP))],
        out_shape=[jax.ShapeDtypeStruct((NP, H), f32),
                   jax.ShapeDtypeStruct((NP, MWP), f32),
                   jax.ShapeDtypeStruct((NP, MWP), f32)],
    )

    node2 = pl.pallas_call(
        _node2_body,
        grid=(gn,),
        in_specs=[_blk2((NC, BN, H)), _blk((BN, H)),
                  _full((H, 2 * H)), _full((H, 2 * H)),
                  _full((H, 2 * H)), _full((H, 2 * H)), _full((1, 2 * H))],
        out_specs=[_blk((BN, 2 * H)), _blk((BN, 2 * H))],
        out_shape=[jax.ShapeDtypeStruct((NP, 2 * H), f32)] * 2,
    )

    econv = pl.pallas_call(
        _econv_body,
        grid=(ge,),
        in_specs=[_blk((BE, MWP)), _blk((BE, MWP)),
                  _full((MWP, H)), _full((1, H))],
        out_specs=_blk((BE, H)),
        out_shape=jax.ShapeDtypeStruct((E, H), f32),
    )

    efin = pl.pallas_call(
        _efin_body,
        grid=(ge,),
        in_specs=[_blk((BE, 2 * H)), _blk((BE, 2 * H)),
                  _full((2 * H, 2 * H)), _full((1, 2 * H)),
                  _full((2 * H, 8)), _full((1, 8))],
        out_specs=_blk((BE, 8)),
        out_shape=jax.ShapeDtypeStruct((E, 8), f32),
    )

    gather2 = _make_gather2(E, MWP)
    scatter = _make_scatter(E, H)

    P0, Q0 = node0(xp, dn, W1p, b1p, W_in2, b2p, W_in3, b3p,
                   A0h, A0x, B0h, B0x, ba0)
    Pg0, Qg0 = gather2(P0, Q0, col, row)
    m0 = econv(Pg0, Qg0, Wb0, bb0)
    Hp0 = scatter(m0, col, zeros_nh)

    H1, P1, Q1 = node1(Hp0, xp, dn, A1h, A1x, B1h, B1x, ba1)
    Pg1, Qg1 = gather2(P1, Q1, col, row)
    m1 = econv(Pg1, Qg1, Wb1, bb1)
    Hp1 = scatter(m1, col, zeros_nh)

    R, S = node2(Hp1, H1, Wr2, Wr1, Ws2, Ws1, be1)
    Rg, Sg = gather2(R, S, row, col)
    out8 = efin(Rg, Sg, W_e2, be2, W3p, b3f)
    return out8[:, :4]


# speedup vs baseline: 3.0998x; 1.7303x over previous
import functools

import jax
import jax.numpy as jnp
from jax import lax
from jax.experimental import pallas as pl
from jax.experimental.pallas import tpu as pltpu
from jax.experimental.pallas import tpu_sc as plsc

f32 = jnp.float32
u32 = jnp.uint32

NP = 10240
MWP = 208
BN = 2048
BE = 2560
NC, NS = 2, 16
NWK = NC * NS
CG = 200
CS = 200


def _elu(v):
    return jnp.where(v > 0, v, jnp.exp(jnp.minimum(v, 0.0)) - 1.0)


def _dotp(a, b):
    return jax.lax.dot_general(a, b, (((1,), (0,)), ((), ())),
                               precision=lax.Precision.HIGHEST,
                               preferred_element_type=f32)


def _dot(a, b):
    return jax.lax.dot_general(a, b, (((1,), (0,)), ((), ())),
                               precision=lax.Precision.DEFAULT,
                               preferred_element_type=f32)


def _bits(v):
    return jax.lax.bitcast_convert_type(v, u32)


def _fl(w):
    return jax.lax.bitcast_convert_type(w, f32)


def _rne_hi(v):
    w = _bits(v)
    w = w + jnp.uint32(0x7FFF) + ((w >> 16) & jnp.uint32(1))
    return w & jnp.uint32(0xFFFF0000)


def _pack(hi, lo):
    return _fl(_rne_hi(hi) | (_rne_hi(lo) >> 16))


def _unpack(packed):
    w = _bits(packed)
    return _fl(w & jnp.uint32(0xFFFF0000)), _fl(w << 16)


def _node0_body(xr, dnr, w1r, b1r, w2r, b2r, w3r, b3r, t_out):
    xn = xr[...] * dnr[...]
    h = jnp.tanh(_dotp(xn[:, :8], w1r[...]) + b1r[...])
    h = jnp.tanh(_dotp(h, w2r[...]) + b2r[...])
    h0 = jnp.tanh(_dotp(h, w3r[...]) + b3r[...])
    t_out[...] = _pack(h0, xn)


def _node1_body(hpr, xr, dnr, h1_out, t_out):
    h1 = hpr[0] + hpr[1]
    xn = xr[...] * dnr[...]
    h1_out[...] = h1
    t_out[...] = _pack(h1, xn)


def _node2_body(hpr, h1r, t_out):
    h2 = hpr[0] + hpr[1]
    t_out[...] = _pack(h2, h1r[...])


def _econv_body(gcr, grr, ahr, axr, bhr, bxr, bar, wbr, bbr, m_out):
    hc, lc = _unpack(gcr[...])
    hr, lr = _unpack(grr[...])
    m1 = (_dot(hc, ahr[...]) + _dot(lc, axr[...]) +
          _dot(hr, bhr[...]) + _dot(lr, bxr[...]) + bar[...])
    u = _elu(m1)
    m_out[...] = _elu(_dot(u, wbr[...]) + bbr[...])


def _efin_body(grr, gcr, w1a, w1b, w1c, w1d, b1r, w2r, b2r, w3r, b3r,
               o_out):
    hr, lr = _unpack(grr[...])
    hc, lc = _unpack(gcr[...])
    e1 = _elu(_dot(hr, w1a[...]) + _dot(lr, w1b[...]) +
              _dot(hc, w1c[...]) + _dot(lc, w1d[...]) + b1r[...])
    e2 = _elu(_dot(e1, w2r[...]) + b2r[...])
    lg = _dot(e2, w3r[...]) + b3r[...]
    mx = jnp.max(lg, axis=-1, keepdims=True)
    sh = lg - mx
    lse = jnp.log(jnp.sum(jnp.exp(sh), axis=-1, keepdims=True))
    o_out[...] = sh - lse


def _full(shape):
    return pl.BlockSpec(shape, lambda i: (0,) * len(shape))


def _blk(shape):
    def im(i):
        return (i,) + (0,) * (len(shape) - 1)
    return pl.BlockSpec(shape, im)


def _blk2(shape):
    def im(i):
        return (0, i) + (0,) * (len(shape) - 2)
    return pl.BlockSpec(shape, im)


def _make_gather2(E, W):
    perw = E // NWK
    nch = perw // CG
    mesh = plsc.VectorSubcoreMesh(core_axis_name="c", subcore_axis_name="s")

    @functools.partial(
        pl.kernel, mesh=mesh,
        out_type=[jax.ShapeDtypeStruct((E, W), f32),
                  jax.ShapeDtypeStruct((E, W), f32)],
        scratch_types=[pltpu.VMEM((CG,), jnp.int32),
                       pltpu.VMEM((CG,), jnp.int32),
                       pltpu.VMEM((CG, W), f32),
                       pltpu.VMEM((CG, W), f32),
                       pltpu.SemaphoreType.DMA,
                       pltpu.SemaphoreType.DMA],
    )
    def gather2(tab, ia, ib, oa, ob, iva, ivb, ra, rb, sema, semb):
        wid = lax.axis_index("s") * NC + lax.axis_index("c")
        base = wid * perw

        def body(i, carry):
            off = pl.multiple_of(base + i * CG, 8)
            pltpu.sync_copy(ia.at[pl.ds(off, CG)], iva)
            pltpu.sync_copy(ib.at[pl.ds(off, CG)], ivb)
            cpa = pltpu.async_copy(tab.at[iva], ra, sema)
            cpb = pltpu.async_copy(tab.at[ivb], rb, semb)
            cpa.wait()
            cpb.wait()
            pltpu.sync_copy(ra, oa.at[pl.ds(off, CG)])
            pltpu.sync_copy(rb, ob.at[pl.ds(off, CG)])
            return carry

        lax.fori_loop(0, nch, body, 0)

    return gather2


def _make_scatter(E, H):
    perw = E // NWK
    nch = perw // CS
    rows_per_tile = NP // NS
    mesh = plsc.VectorSubcoreMesh(core_axis_name="c", subcore_axis_name="s")

    @functools.partial(
        pl.kernel, mesh=mesh,
        out_type=jax.ShapeDtypeStruct((NC, NP, H), f32),
        scratch_types=[pltpu.VMEM((CS,), jnp.int32),
                       pltpu.VMEM((CS, H), f32),
                       pltpu.VMEM_SHARED((NP, H), f32)],
    )
    def scat(mref, cref, zref, out, iv, rv, acc):
        c = lax.axis_index("c")
        s = lax.axis_index("s")
        r0 = s * rows_per_tile
        pltpu.sync_copy(zref.at[pl.ds(r0, rows_per_tile)],
                        acc.at[pl.ds(r0, rows_per_tile)])
        plsc.subcore_barrier()
        base = (s * NC + c) * perw

        def body(i, carry):
            off = pl.multiple_of(base + i * CS, 8)
            pltpu.sync_copy(cref.at[pl.ds(off, CS)], iv)
            pltpu.sync_copy(mref.at[pl.ds(off, CS)], rv)
            pltpu.sync_copy(rv, acc.at[iv], add=True)
            return carry

        lax.fori_loop(0, nch, body, 0)
        plsc.subcore_barrier()
        pltpu.sync_copy(acc.at[pl.ds(r0, rows_per_tile)],
                        out.at[c, pl.ds(r0, rows_per_tile)])

    return scat


def kernel(x, edge_index, datanorm, W_in1, b_in1, W_in2, b_in2, W_in3,
           b_in3, W_c0a, b_c0a, W_c0b, b_c0b, W_c1a, b_c1a, W_c1b, b_c1b,
           W_e1, b_e1, W_e2, b_e2, W_e3, b_e3):
    N, D = x.shape
    E = edge_index.shape[1]
    H = W_in3.shape[1]
    F = H + D

    row = edge_index[0]
    col = edge_index[1]

    def pad2(w, r, c):
        return jnp.zeros((r, c), f32).at[:w.shape[0], :w.shape[1]].set(w)

    xp = jnp.zeros((NP, H), f32).at[:N, :D].set(x)
    dn = jnp.zeros((1, H), f32).at[0, :D].set(datanorm)
    zeros_nh = jnp.zeros((NP, H), f32)

    W1p = pad2(W_in1, 8, 2 * H)
    b1p = b_in1[None]
    b2p = b_in2[None]
    b3p = b_in3[None]

    def conv_prep(Wa, ba, Wb, bb):
        A = Wa[:F] - Wa[F:]
        B = Wa[F:]
        return (pad2(A[:H], H, MWP), pad2(A[H:], H, MWP),
                pad2(B[:H], H, MWP), pad2(B[H:], H, MWP),
                pad2(ba[None], 1, MWP), pad2(Wb, MWP, H), bb[None])

    A0h, A0x, B0h, B0x, ba0, Wb0, bb0 = conv_prep(W_c0a, b_c0a, W_c0b, b_c0b)
    A1h, A1x, B1h, B1x, ba1, Wb1, bb1 = conv_prep(W_c1a, b_c1a, W_c1b, b_c1b)

    W1a, W1b = W_e1[:H], W_e1[H:2 * H]
    W1c, W1d = W_e1[2 * H:3 * H], W_e1[3 * H:]
    be1 = b_e1[None]
    be2 = b_e2[None]
    W3p = pad2(W_e3, 2 * H, 8)
    b3f = jnp.full((1, 8), -1e30, f32).at[0, :W_e3.shape[1]].set(b_e3)

    gn = NP // BN
    ge = E // BE

    node0 = pl.pallas_call(
        _node0_body,
        grid=(gn,),
        in_specs=[_blk((BN, H)), _full((1, H)),
                  _full((8, 2 * H)), _full((1, 2 * H)),
                  _full((2 * H, 2 * H)), _full((1, 2 * H)),
                  _full((2 * H, H)), _full((1, H))],
        out_specs=_blk((BN, H)),
        out_shape=jax.ShapeDtypeStruct((NP, H), f32),
    )

    node1 = pl.pallas_call(
        _node1_body,
        grid=(gn,),
        in_specs=[_blk2((NC, BN, H)), _blk((BN, H)), _full((1, H))],
        out_specs=[_blk((BN, H)), _blk((BN, H))],
        out_shape=[jax.ShapeDtypeStruct((NP, H), f32),
                   jax.ShapeDtypeStruct((NP, H), f32)],
    )

    node2 = pl.pallas_call(
        _node2_body,
        grid=(gn,),
        in_specs=[_blk2((NC, BN, H)), _blk((BN, H))],
        out_specs=_blk((BN, H)),
        out_shape=jax.ShapeDtypeStruct((NP, H), f32),
    )

    econv = pl.pallas_call(
        _econv_body,
        grid=(ge,),
        in_specs=[_blk((BE, H)), _blk((BE, H)),
                  _full((H, MWP)), _full((H, MWP)),
                  _full((H, MWP)), _full((H, MWP)), _full((1, MWP)),
                  _full((MWP, H)), _full((1, H))],
        out_specs=_blk((BE, H)),
        out_shape=jax.ShapeDtypeStruct((E, H), f32),
    )

    efin = pl.pallas_call(
        _efin_body,
        grid=(ge,),
        in_specs=[_blk((BE, H)), _blk((BE, H)),
                  _full((H, 2 * H)), _full((H, 2 * H)),
                  _full((H, 2 * H)), _full((H, 2 * H)), _full((1, 2 * H)),
                  _full((2 * H, 2 * H)), _full((1, 2 * H)),
                  _full((2 * H, 8)), _full((1, 8))],
        out_specs=_blk((BE, 8)),
        out_shape=jax.ShapeDtypeStruct((E, 8), f32),
    )

    gather2 = _make_gather2(E, H)
    scatter = _make_scatter(E, H)

    T0 = node0(xp, dn, W1p, b1p, W_in2, b2p, W_in3, b3p)
    Gc0, Gr0 = gather2(T0, col, row)
    m0 = econv(Gc0, Gr0, A0h, A0x, B0h, B0x, ba0, Wb0, bb0)
    Hp0 = scatter(m0, col, zeros_nh)

    H1, T1 = node1(Hp0, xp, dn)
    Gc1, Gr1 = gather2(T1, col, row)
    m1 = econv(Gc1, Gr1, A1h, A1x, B1h, B1x, ba1, Wb1, bb1)
    Hp1 = scatter(m1, col, zeros_nh)

    T2 = node2(Hp1, H1)
    Gr2, Gc2 = gather2(T2, row, col)
    out8 = efin(Gr2, Gc2, W1a, W1b, W1c, W1d, be1, W_e2, be2, W3p, b3f)
    return out8[:, :4]


# speedup vs baseline: 3.3785x; 1.0899x over previous
import functools

import jax
import jax.numpy as jnp
from jax import lax
from jax.experimental import pallas as pl
from jax.experimental.pallas import tpu as pltpu
from jax.experimental.pallas import tpu_sc as plsc

f32 = jnp.float32
u32 = jnp.uint32

NP = 10240
MWP = 208
BN = 2048
BE = 2560
NC, NS = 2, 16
NWK = NC * NS
CG = 200
CS = 200


def _elu(v):
    return jnp.where(v > 0, v, jnp.exp(v) - 1.0)


def _dotp(a, b):
    return jax.lax.dot_general(a, b, (((1,), (0,)), ((), ())),
                               precision=lax.Precision.HIGHEST,
                               preferred_element_type=f32)


def _dot(a, b):
    return jax.lax.dot_general(a, b, (((1,), (0,)), ((), ())),
                               precision=lax.Precision.DEFAULT,
                               preferred_element_type=f32)


def _bits(v):
    return jax.lax.bitcast_convert_type(v, u32)


def _fl(w):
    return jax.lax.bitcast_convert_type(w, f32)


def _rne_hi(v):
    w = _bits(v)
    w = w + jnp.uint32(0x7FFF) + ((w >> 16) & jnp.uint32(1))
    return w & jnp.uint32(0xFFFF0000)


def _pack(hi, lo):
    return _fl(_rne_hi(hi) | (_rne_hi(lo) >> 16))


def _unpack(packed):
    w = _bits(packed)
    return jnp.concatenate(
        [_fl(w & jnp.uint32(0xFFFF0000)), _fl(w << 16)], axis=1)


def _node0_body(xr, dnr, w1r, b1r, w2r, b2r, w3r, b3r, t_out):
    xn = xr[...] * dnr[...]
    h = jnp.tanh(_dotp(xn[:, :8], w1r[...]) + b1r[...])
    h = jnp.tanh(_dotp(h, w2r[...]) + b2r[...])
    h0 = jnp.tanh(_dotp(h, w3r[...]) + b3r[...])
    t_out[...] = _pack(h0, xn)


def _node1_body(hpr, xr, dnr, h1_out, t_out):
    h1 = hpr[0] + hpr[1]
    xn = xr[...] * dnr[...]
    h1_out[...] = h1
    t_out[...] = _pack(h1, xn)


def _node2_body(hpr, h1r, t_out):
    h2 = hpr[0] + hpr[1]
    t_out[...] = _pack(h2, h1r[...])


def _econv_body(gcr, grr, aar, bbr2, bar, wbr, bbr, m_out):
    fc = _unpack(gcr[...])
    fr = _unpack(grr[...])
    m1 = _dot(fc, aar[...]) + _dot(fr, bbr2[...]) + bar[...]
    u = _elu(m1)
    m_out[...] = _elu(_dot(u, wbr[...]) + bbr[...])


def _efin_body(grr, gcr, w1a, w1c, b1r, w2r, b2r, w3r, b3r, o_out):
    fr = _unpack(grr[...])
    fc = _unpack(gcr[...])
    e1 = _elu(_dot(fr, w1a[...]) + _dot(fc, w1c[...]) + b1r[...])
    e2 = _elu(_dot(e1, w2r[...]) + b2r[...])
    lg = _dot(e2, w3r[...]) + b3r[...]
    mx = jnp.max(lg, axis=-1, keepdims=True)
    sh = lg - mx
    lse = jnp.log(jnp.sum(jnp.exp(sh), axis=-1, keepdims=True))
    o_out[...] = sh - lse


def _full(shape):
    return pl.BlockSpec(shape, lambda i: (0,) * len(shape))


def _blk(shape):
    def im(i):
        return (i,) + (0,) * (len(shape) - 1)
    return pl.BlockSpec(shape, im)


def _blk2(shape):
    def im(i):
        return (0, i) + (0,) * (len(shape) - 2)
    return pl.BlockSpec(shape, im)


def _make_gather2(E, W):
    perw = E // NWK
    nch = perw // CG
    mesh = plsc.VectorSubcoreMesh(core_axis_name="c", subcore_axis_name="s")

    @functools.partial(
        pl.kernel, mesh=mesh,
        out_type=[jax.ShapeDtypeStruct((E, W), f32),
                  jax.ShapeDtypeStruct((E, W), f32)],
        scratch_types=[pltpu.VMEM((CG,), jnp.int32),
                       pltpu.VMEM((CG,), jnp.int32),
                       pltpu.VMEM((CG, W), f32),
                       pltpu.VMEM((CG, W), f32),
                       pltpu.SemaphoreType.DMA,
                       pltpu.SemaphoreType.DMA],
    )
    def gather2(tab, ia, ib, oa, ob, iva, ivb, ra, rb, sema, semb):
        wid = lax.axis_index("s") * NC + lax.axis_index("c")
        base = wid * perw

        def body(i, carry):
            off = pl.multiple_of(base + i * CG, 8)
            pltpu.sync_copy(ia.at[pl.ds(off, CG)], iva)
            pltpu.sync_copy(ib.at[pl.ds(off, CG)], ivb)
            cpa = pltpu.async_copy(tab.at[iva], ra, sema)
            cpb = pltpu.async_copy(tab.at[ivb], rb, semb)
            cpa.wait()
            cpb.wait()
            pltpu.sync_copy(ra, oa.at[pl.ds(off, CG)])
            pltpu.sync_copy(rb, ob.at[pl.ds(off, CG)])
            return carry

        lax.fori_loop(0, nch, body, 0)

    return gather2


def _make_scatter(E, H):
    perw = E // NWK
    nch = perw // CS
    rows_per_tile = NP // NS
    mesh = plsc.VectorSubcoreMesh(core_axis_name="c", subcore_axis_name="s")

    @functools.partial(
        pl.kernel, mesh=mesh,
        out_type=jax.ShapeDtypeStruct((NC, NP, H), f32),
        scratch_types=[pltpu.VMEM((CS,), jnp.int32),
                       pltpu.VMEM((CS, H), f32),
                       pltpu.VMEM_SHARED((NP, H), f32)],
    )
    def scat(mref, cref, zref, out, iv, rv, acc):
        c = lax.axis_index("c")
        s = lax.axis_index("s")
        r0 = s * rows_per_tile
        pltpu.sync_copy(zref.at[pl.ds(r0, rows_per_tile)],
                        acc.at[pl.ds(r0, rows_per_tile)])
        plsc.subcore_barrier()
        base = (s * NC + c) * perw

        def body(i, carry):
            off = pl.multiple_of(base + i * CS, 8)
            pltpu.sync_copy(cref.at[pl.ds(off, CS)], iv)
            pltpu.sync_copy(mref.at[pl.ds(off, CS)], rv)
            pltpu.sync_copy(rv, acc.at[iv], add=True)
            return carry

        lax.fori_loop(0, nch, body, 0)
        plsc.subcore_barrier()
        pltpu.sync_copy(acc.at[pl.ds(r0, rows_per_tile)],
                        out.at[c, pl.ds(r0, rows_per_tile)])

    return scat


def kernel(x, edge_index, datanorm, W_in1, b_in1, W_in2, b_in2, W_in3,
           b_in3, W_c0a, b_c0a, W_c0b, b_c0b, W_c1a, b_c1a, W_c1b, b_c1b,
           W_e1, b_e1, W_e2, b_e2, W_e3, b_e3):
    N, D = x.shape
    E = edge_index.shape[1]
    H = W_in3.shape[1]
    F = H + D

    row = edge_index[0]
    col = edge_index[1]

    def pad2(w, r, c):
        return jnp.zeros((r, c), f32).at[:w.shape[0], :w.shape[1]].set(w)

    xp = jnp.zeros((NP, H), f32).at[:N, :D].set(x)
    dn = jnp.zeros((1, H), f32).at[0, :D].set(datanorm)
    zeros_nh = jnp.zeros((NP, H), f32)

    W1p = pad2(W_in1, 8, 2 * H)
    b1p = b_in1[None]
    b2p = b_in2[None]
    b3p = b_in3[None]

    def conv_prep(Wa, ba, Wb, bb):
        A = Wa[:F] - Wa[F:]
        B = Wa[F:]
        return (pad2(A, 2 * H, MWP), pad2(B, 2 * H, MWP),
                pad2(ba[None], 1, MWP), pad2(Wb, MWP, H), bb[None])

    AA0, BB0, ba0, Wb0, bb0 = conv_prep(W_c0a, b_c0a, W_c0b, b_c0b)
    AA1, BB1, ba1, Wb1, bb1 = conv_prep(W_c1a, b_c1a, W_c1b, b_c1b)

    W1a, W1c = W_e1[:2 * H], W_e1[2 * H:]
    be1 = b_e1[None]
    be2 = b_e2[None]
    W3p = pad2(W_e3, 2 * H, 8)
    b3f = jnp.full((1, 8), -1e30, f32).at[0, :W_e3.shape[1]].set(b_e3)

    gn = NP // BN
    ge = E // BE

    node0 = pl.pallas_call(
        _node0_body,
        grid=(gn,),
        in_specs=[_blk((BN, H)), _full((1, H)),
                  _full((8, 2 * H)), _full((1, 2 * H)),
                  _full((2 * H, 2 * H)), _full((1, 2 * H)),
                  _full((2 * H, H)), _full((1, H))],
        out_specs=_blk((BN, H)),
        out_shape=jax.ShapeDtypeStruct((NP, H), f32),
    )

    node1 = pl.pallas_call(
        _node1_body,
        grid=(gn,),
        in_specs=[_blk2((NC, BN, H)), _blk((BN, H)), _full((1, H))],
        out_specs=[_blk((BN, H)), _blk((BN, H))],
        out_shape=[jax.ShapeDtypeStruct((NP, H), f32),
                   jax.ShapeDtypeStruct((NP, H), f32)],
    )

    node2 = pl.pallas_call(
        _node2_body,
        grid=(gn,),
        in_specs=[_blk2((NC, BN, H)), _blk((BN, H))],
        out_specs=_blk((BN, H)),
        out_shape=jax.ShapeDtypeStruct((NP, H), f32),
    )

    econv = pl.pallas_call(
        _econv_body,
        grid=(ge,),
        in_specs=[_blk((BE, H)), _blk((BE, H)),
                  _full((2 * H, MWP)), _full((2 * H, MWP)), _full((1, MWP)),
                  _full((MWP, H)), _full((1, H))],
        out_specs=_blk((BE, H)),
        out_shape=jax.ShapeDtypeStruct((E, H), f32),
    )

    efin = pl.pallas_call(
        _efin_body,
        grid=(ge,),
        in_specs=[_blk((BE, H)), _blk((BE, H)),
                  _full((2 * H, 2 * H)), _full((2 * H, 2 * H)),
                  _full((1, 2 * H)),
                  _full((2 * H, 2 * H)), _full((1, 2 * H)),
                  _full((2 * H, 8)), _full((1, 8))],
        out_specs=_blk((BE, 8)),
        out_shape=jax.ShapeDtypeStruct((E, 8), f32),
    )

    gather2 = _make_gather2(E, H)
    scatter = _make_scatter(E, H)

    T0 = node0(xp, dn, W1p, b1p, W_in2, b2p, W_in3, b3p)
    Gc0, Gr0 = gather2(T0, col, row)
    m0 = econv(Gc0, Gr0, AA0, BB0, ba0, Wb0, bb0)
    Hp0 = scatter(m0, col, zeros_nh)

    H1, T1 = node1(Hp0, xp, dn)
    Gc1, Gr1 = gather2(T1, col, row)
    m1 = econv(Gc1, Gr1, AA1, BB1, ba1, Wb1, bb1)
    Hp1 = scatter(m1, col, zeros_nh)

    T2 = node2(Hp1, H1)
    Gr2, Gc2 = gather2(T2, row, col)
    out8 = efin(Gr2, Gc2, W1a, W1c, be1, W_e2, be2, W3p, b3f)
    return out8[:, :4]


# speedup vs baseline: 3.7848x; 1.1203x over previous
import functools

import jax
import jax.numpy as jnp
from jax import lax
from jax.experimental import pallas as pl
from jax.experimental.pallas import tpu as pltpu
from jax.experimental.pallas import tpu_sc as plsc

f32 = jnp.float32
u32 = jnp.uint32

NP = 10240
MWP = 208
BN = 2048
BE = 2560
NC, NS = 2, 16
NWK = NC * NS
CG = 200
CS = 200


def _elu(v):
    return jnp.where(v > 0, v, jnp.exp(v) - 1.0)


def _dotp(a, b):
    return jax.lax.dot_general(a, b, (((1,), (0,)), ((), ())),
                               precision=lax.Precision.HIGHEST,
                               preferred_element_type=f32)


def _dot(a, b):
    return jax.lax.dot_general(a, b, (((1,), (0,)), ((), ())),
                               precision=lax.Precision.DEFAULT,
                               preferred_element_type=f32)


def _bits(v):
    return jax.lax.bitcast_convert_type(v, u32)


def _fl(w):
    return jax.lax.bitcast_convert_type(w, f32)


def _rne_hi(v):
    w = _bits(v)
    w = w + jnp.uint32(0x7FFF) + ((w >> 16) & jnp.uint32(1))
    return w & jnp.uint32(0xFFFF0000)


def _pack(hi, lo):
    return _fl(_rne_hi(hi) | (_rne_hi(lo) >> 16))


def _unpack(packed):
    w = _bits(packed)
    return jnp.concatenate(
        [_fl(w & jnp.uint32(0xFFFF0000)), _fl(w << 16)], axis=1)


def _node0_body(xr, dnr, w1r, b1r, w2r, b2r, w3r, b3r, t_out):
    xn = xr[...] * dnr[...]
    h = jnp.tanh(_dotp(xn[:, :8], w1r[...]) + b1r[...])
    h = jnp.tanh(_dotp(h, w2r[...]) + b2r[...])
    h0 = jnp.tanh(_dotp(h, w3r[...]) + b3r[...])
    t_out[...] = _pack(h0, xn)


def _node1_body(hpr, xr, dnr, h1_out, t_out):
    h1 = hpr[0] + hpr[1]
    xn = xr[...] * dnr[...]
    h1_out[...] = h1
    t_out[...] = _pack(h1, xn)


def _node2_body(hpr, h1r, t_out):
    h2 = hpr[0] + hpr[1]
    t_out[...] = _pack(h2, h1r[...])


def _econv_body(gcr, grr, aar, bbr2, bar, wbr, bbr, m_out):
    fc = _unpack(gcr[...])
    fr = _unpack(grr[...])
    m1 = _dot(fc, aar[...]) + _dot(fr, bbr2[...]) + bar[...]
    u = _elu(m1)
    m_out[...] = _elu(_dot(u, wbr[...]) + bbr[...])


def _efin_body(grr, gcr, w1a, w1c, b1r, w2r, b2r, w3r, b3r, o_out):
    fr = _unpack(grr[...])
    fc = _unpack(gcr[...])
    e1 = _elu(_dot(fr, w1a[...]) + _dot(fc, w1c[...]) + b1r[...])
    e2 = _elu(_dot(e1, w2r[...]) + b2r[...])
    lg = _dot(e2, w3r[...]) + b3r[...]
    mx = jnp.max(lg, axis=-1, keepdims=True)
    sh = lg - mx
    lse = jnp.log(jnp.sum(jnp.exp(sh), axis=-1, keepdims=True))
    o_out[...] = sh - lse


def _full(shape):
    return pl.BlockSpec(shape, lambda i: (0,) * len(shape))


def _blk(shape):
    def im(i):
        return (i,) + (0,) * (len(shape) - 1)
    return pl.BlockSpec(shape, im)


def _blk2(shape):
    def im(i):
        return (0, i) + (0,) * (len(shape) - 2)
    return pl.BlockSpec(shape, im)


def _make_gather2(E, W):
    perw = E // NWK
    nch = perw // CG
    mesh = plsc.VectorSubcoreMesh(core_axis_name="c", subcore_axis_name="s")

    @functools.partial(
        pl.kernel, mesh=mesh,
        out_type=[jax.ShapeDtypeStruct((E, W), f32),
                  jax.ShapeDtypeStruct((E, W), f32)],
        scratch_types=[pltpu.VMEM((CG,), jnp.int32),
                       pltpu.VMEM((CG,), jnp.int32),
                       pltpu.VMEM((CG,), jnp.int32),
                       pltpu.VMEM((CG,), jnp.int32),
                       pltpu.VMEM((CG, W), f32),
                       pltpu.VMEM((CG, W), f32),
                       pltpu.VMEM((CG, W), f32),
                       pltpu.VMEM((CG, W), f32),
                       pltpu.SemaphoreType.DMA, pltpu.SemaphoreType.DMA,
                       pltpu.SemaphoreType.DMA, pltpu.SemaphoreType.DMA,
                       pltpu.SemaphoreType.DMA, pltpu.SemaphoreType.DMA,
                       pltpu.SemaphoreType.DMA, pltpu.SemaphoreType.DMA],
    )
    def gather2(tab, ia, ib, oa, ob, iva0, ivb0, iva1, ivb1,
                ra0, rb0, ra1, rb1, ga0, gb0, ga1, gb1,
                wa0, wb0, wa1, wb1):
        wid = lax.axis_index("s") * NC + lax.axis_index("c")
        base = wid * perw

        def body(p, carry):
            c0 = pl.multiple_of(base + (2 * p) * CG, 8)
            c1 = pl.multiple_of(base + (2 * p + 1) * CG, 8)
            pltpu.sync_copy(ia.at[pl.ds(c0, CG)], iva0)
            pltpu.sync_copy(ib.at[pl.ds(c0, CG)], ivb0)
            cpa0 = pltpu.async_copy(tab.at[iva0], ra0, ga0)
            cpb0 = pltpu.async_copy(tab.at[ivb0], rb0, gb0)
            pltpu.sync_copy(ia.at[pl.ds(c1, CG)], iva1)
            pltpu.sync_copy(ib.at[pl.ds(c1, CG)], ivb1)
            cpa1 = pltpu.async_copy(tab.at[iva1], ra1, ga1)
            cpb1 = pltpu.async_copy(tab.at[ivb1], rb1, gb1)
            cpa0.wait()
            swa0 = pltpu.async_copy(ra0, oa.at[pl.ds(c0, CG)], wa0)
            cpb0.wait()
            swb0 = pltpu.async_copy(rb0, ob.at[pl.ds(c0, CG)], wb0)
            cpa1.wait()
            swa1 = pltpu.async_copy(ra1, oa.at[pl.ds(c1, CG)], wa1)
            cpb1.wait()
            swb1 = pltpu.async_copy(rb1, ob.at[pl.ds(c1, CG)], wb1)
            swa0.wait()
            swb0.wait()
            swa1.wait()
            swb1.wait()
            return carry

        lax.fori_loop(0, nch // 2, body, 0)

    return gather2


def _make_scatter(E, H):
    perw = E // NWK
    nch = perw // CS
    rows_per_tile = NP // NS
    mesh = plsc.VectorSubcoreMesh(core_axis_name="c", subcore_axis_name="s")

    @functools.partial(
        pl.kernel, mesh=mesh,
        out_type=jax.ShapeDtypeStruct((NC, NP, H), f32),
        scratch_types=[pltpu.VMEM((CS,), jnp.int32),
                       pltpu.VMEM((CS, H), f32),
                       pltpu.SemaphoreType.DMA, pltpu.SemaphoreType.DMA,
                       pltpu.SemaphoreType.DMA,
                       pltpu.VMEM_SHARED((NP, H), f32)],
    )
    def scat(mref, cref, zref, out, iv, rv, li, lr, sa, acc):
        c = lax.axis_index("c")
        s = lax.axis_index("s")
        r0 = s * rows_per_tile
        pltpu.sync_copy(zref.at[pl.ds(r0, rows_per_tile)],
                        acc.at[pl.ds(r0, rows_per_tile)])
        plsc.subcore_barrier()
        base = (s * NC + c) * perw

        def body(i, carry):
            off = pl.multiple_of(base + i * CS, 8)
            hi = pltpu.async_copy(cref.at[pl.ds(off, CS)], iv, li)
            hr = pltpu.async_copy(mref.at[pl.ds(off, CS)], rv, lr)
            hi.wait()
            hr.wait()
            pltpu.async_copy(rv, acc.at[iv], sa, add=True).wait()
            return carry

        lax.fori_loop(0, nch, body, 0)
        plsc.subcore_barrier()
        pltpu.sync_copy(acc.at[pl.ds(r0, rows_per_tile)],
                        out.at[c, pl.ds(r0, rows_per_tile)])

    return scat


def kernel(x, edge_index, datanorm, W_in1, b_in1, W_in2, b_in2, W_in3,
           b_in3, W_c0a, b_c0a, W_c0b, b_c0b, W_c1a, b_c1a, W_c1b, b_c1b,
           W_e1, b_e1, W_e2, b_e2, W_e3, b_e3):
    N, D = x.shape
    E = edge_index.shape[1]
    H = W_in3.shape[1]
    F = H + D

    row = edge_index[0]
    col = edge_index[1]

    def pad2(w, r, c):
        return jnp.zeros((r, c), f32).at[:w.shape[0], :w.shape[1]].set(w)

    xp = jnp.zeros((NP, H), f32).at[:N, :D].set(x)
    dn = jnp.zeros((1, H), f32).at[0, :D].set(datanorm)
    zeros_nh = jnp.zeros((NP, H), f32)

    W1p = pad2(W_in1, 8, 2 * H)
    b1p = b_in1[None]
    b2p = b_in2[None]
    b3p = b_in3[None]

    def conv_prep(Wa, ba, Wb, bb):
        A = Wa[:F] - Wa[F:]
        B = Wa[F:]
        return (pad2(A, 2 * H, MWP), pad2(B, 2 * H, MWP),
                pad2(ba[None], 1, MWP), pad2(Wb, MWP, H), bb[None])

    AA0, BB0, ba0, Wb0, bb0 = conv_prep(W_c0a, b_c0a, W_c0b, b_c0b)
    AA1, BB1, ba1, Wb1, bb1 = conv_prep(W_c1a, b_c1a, W_c1b, b_c1b)

    W1a, W1c = W_e1[:2 * H], W_e1[2 * H:]
    be1 = b_e1[None]
    be2 = b_e2[None]
    W3p = pad2(W_e3, 2 * H, 8)
    b3f = jnp.full((1, 8), -1e30, f32).at[0, :W_e3.shape[1]].set(b_e3)

    gn = NP // BN
    ge = E // BE

    node0 = pl.pallas_call(
        _node0_body,
        grid=(gn,),
        in_specs=[_blk((BN, H)), _full((1, H)),
                  _full((8, 2 * H)), _full((1, 2 * H)),
                  _full((2 * H, 2 * H)), _full((1, 2 * H)),
                  _full((2 * H, H)), _full((1, H))],
        out_specs=_blk((BN, H)),
        out_shape=jax.ShapeDtypeStruct((NP, H), f32),
    )

    node1 = pl.pallas_call(
        _node1_body,
        grid=(gn,),
        in_specs=[_blk2((NC, BN, H)), _blk((BN, H)), _full((1, H))],
        out_specs=[_blk((BN, H)), _blk((BN, H))],
        out_shape=[jax.ShapeDtypeStruct((NP, H), f32),
                   jax.ShapeDtypeStruct((NP, H), f32)],
    )

    node2 = pl.pallas_call(
        _node2_body,
        grid=(gn,),
        in_specs=[_blk2((NC, BN, H)), _blk((BN, H))],
        out_specs=_blk((BN, H)),
        out_shape=jax.ShapeDtypeStruct((NP, H), f32),
    )

    econv = pl.pallas_call(
        _econv_body,
        grid=(ge,),
        in_specs=[_blk((BE, H)), _blk((BE, H)),
                  _full((2 * H, MWP)), _full((2 * H, MWP)), _full((1, MWP)),
                  _full((MWP, H)), _full((1, H))],
        out_specs=_blk((BE, H)),
        out_shape=jax.ShapeDtypeStruct((E, H), f32),
    )

    efin = pl.pallas_call(
        _efin_body,
        grid=(ge,),
        in_specs=[_blk((BE, H)), _blk((BE, H)),
                  _full((2 * H, 2 * H)), _full((2 * H, 2 * H)),
                  _full((1, 2 * H)),
                  _full((2 * H, 2 * H)), _full((1, 2 * H)),
                  _full((2 * H, 8)), _full((1, 8))],
        out_specs=_blk((BE, 8)),
        out_shape=jax.ShapeDtypeStruct((E, 8), f32),
    )

    gather2 = _make_gather2(E, H)
    scatter = _make_scatter(E, H)

    T0 = node0(xp, dn, W1p, b1p, W_in2, b2p, W_in3, b3p)
    Gc0, Gr0 = gather2(T0, col, row)
    m0 = econv(Gc0, Gr0, AA0, BB0, ba0, Wb0, bb0)
    Hp0 = scatter(m0, col, zeros_nh)

    H1, T1 = node1(Hp0, xp, dn)
    Gc1, Gr1 = gather2(T1, col, row)
    m1 = econv(Gc1, Gr1, AA1, BB1, ba1, Wb1, bb1)
    Hp1 = scatter(m1, col, zeros_nh)

    T2 = node2(Hp1, H1)
    Gr2, Gc2 = gather2(T2, row, col)
    out8 = efin(Gr2, Gc2, W1a, W1c, be1, W_e2, be2, W3p, b3f)
    return out8[:, :4]


# speedup vs baseline: 4.1899x; 1.1070x over previous
import functools

import jax
import jax.numpy as jnp
from jax import lax
from jax.experimental import pallas as pl
from jax.experimental.pallas import tpu as pltpu
from jax.experimental.pallas import tpu_sc as plsc

f32 = jnp.float32
u32 = jnp.uint32

NP = 10240
MWP = 208
BN = 2048
BE = 3200
NC, NS = 2, 16
NWK = NC * NS
CG = 200
CS = 200


def _elu(v):
    return jnp.where(v > 0, v, jnp.exp(v) - 1.0)


def _dotp(a, b):
    return jax.lax.dot_general(a, b, (((1,), (0,)), ((), ())),
                               precision=lax.Precision.HIGHEST,
                               preferred_element_type=f32)


def _dot(a, b):
    return jax.lax.dot_general(a, b, (((1,), (0,)), ((), ())),
                               precision=lax.Precision.DEFAULT,
                               preferred_element_type=f32)


def _bits(v):
    return jax.lax.bitcast_convert_type(v, u32)


def _fl(w):
    return jax.lax.bitcast_convert_type(w, f32)


def _rne_hi(v):
    w = _bits(v)
    w = w + jnp.uint32(0x7FFF) + ((w >> 16) & jnp.uint32(1))
    return w & jnp.uint32(0xFFFF0000)


def _pack(hi, lo):
    return _fl(_rne_hi(hi) | (_rne_hi(lo) >> 16))


def _unpack(packed):
    w = _bits(packed)
    return jnp.concatenate(
        [_fl(w & jnp.uint32(0xFFFF0000)), _fl(w << 16)], axis=1)


def _node0_body(xr, dnr, w1r, b1r, w2r, b2r, w3r, b3r, t_out):
    xn = xr[...] * dnr[...]
    h = jnp.tanh(_dotp(xn[:, :8], w1r[...]) + b1r[...])
    h = jnp.tanh(_dotp(h, w2r[...]) + b2r[...])
    h0 = jnp.tanh(_dotp(h, w3r[...]) + b3r[...])
    t_out[...] = _pack(h0, xn)


def _node1_body(hpa, hpb, xr, dnr, h1_out, t_out):
    h1 = (hpa[0] + hpa[1]) + (hpb[0] + hpb[1])
    xn = xr[...] * dnr[...]
    h1_out[...] = h1
    t_out[...] = _pack(h1, xn)


def _node2_body(hpa, hpb, h1r, t_out):
    h2 = (hpa[0] + hpa[1]) + (hpb[0] + hpb[1])
    t_out[...] = _pack(h2, h1r[...])


def _econv_body(gcr, grr, aar, bbr2, bar, wbr, bbr, m_out):
    fc = _unpack(gcr[...])
    fr = _unpack(grr[...])
    m1 = _dot(fc, aar[...]) + _dot(fr, bbr2[...]) + bar[...]
    u = _elu(m1)
    m_out[...] = _elu(_dot(u, wbr[...]) + bbr[...])


def _efin_body(grr, gcr, w1a, w1c, b1r, w2r, b2r, w3r, b3r, o_out):
    fr = _unpack(grr[...])
    fc = _unpack(gcr[...])
    e1 = _elu(_dot(fr, w1a[...]) + _dot(fc, w1c[...]) + b1r[...])
    e2 = _elu(_dot(e1, w2r[...]) + b2r[...])
    lg = _dot(e2, w3r[...]) + b3r[...]
    mx = jnp.max(lg, axis=-1, keepdims=True)
    sh = lg - mx
    lse = jnp.log(jnp.sum(jnp.exp(sh), axis=-1, keepdims=True))
    o_out[...] = sh - lse


def _full(shape):
    return pl.BlockSpec(shape, lambda i: (0,) * len(shape))


def _blk(shape):
    def im(i):
        return (i,) + (0,) * (len(shape) - 1)
    return pl.BlockSpec(shape, im)


def _blk2(shape):
    def im(i):
        return (0, i) + (0,) * (len(shape) - 2)
    return pl.BlockSpec(shape, im)


def _make_gather2(E, W):
    perw = E // NWK
    nch = perw // CG
    mesh = plsc.VectorSubcoreMesh(core_axis_name="c", subcore_axis_name="s")

    @functools.partial(
        pl.kernel, mesh=mesh,
        out_type=[jax.ShapeDtypeStruct((E, W), f32),
                  jax.ShapeDtypeStruct((E, W), f32)],
        scratch_types=[pltpu.VMEM((CG,), jnp.int32),
                       pltpu.VMEM((CG,), jnp.int32),
                       pltpu.VMEM((CG,), jnp.int32),
                       pltpu.VMEM((CG,), jnp.int32),
                       pltpu.VMEM((CG, W), f32),
                       pltpu.VMEM((CG, W), f32),
                       pltpu.VMEM((CG, W), f32),
                       pltpu.VMEM((CG, W), f32),
                       pltpu.SemaphoreType.DMA, pltpu.SemaphoreType.DMA,
                       pltpu.SemaphoreType.DMA, pltpu.SemaphoreType.DMA,
                       pltpu.SemaphoreType.DMA, pltpu.SemaphoreType.DMA,
                       pltpu.SemaphoreType.DMA, pltpu.SemaphoreType.DMA],
    )
    def gather2(tab, ia, ib, oa, ob, iva0, ivb0, iva1, ivb1,
                ra0, rb0, ra1, rb1, ga0, gb0, ga1, gb1,
                wa0, wb0, wa1, wb1):
        wid = lax.axis_index("s") * NC + lax.axis_index("c")
        base = wid * perw

        def body(p, carry):
            c0 = pl.multiple_of(base + (2 * p) * CG, 8)
            c1 = pl.multiple_of(base + (2 * p + 1) * CG, 8)
            pltpu.sync_copy(ia.at[pl.ds(c0, CG)], iva0)
            pltpu.sync_copy(ib.at[pl.ds(c0, CG)], ivb0)
            cpa0 = pltpu.async_copy(tab.at[iva0], ra0, ga0)
            cpb0 = pltpu.async_copy(tab.at[ivb0], rb0, gb0)
            pltpu.sync_copy(ia.at[pl.ds(c1, CG)], iva1)
            pltpu.sync_copy(ib.at[pl.ds(c1, CG)], ivb1)
            cpa1 = pltpu.async_copy(tab.at[iva1], ra1, ga1)
            cpb1 = pltpu.async_copy(tab.at[ivb1], rb1, gb1)
            cpa0.wait()
            swa0 = pltpu.async_copy(ra0, oa.at[pl.ds(c0, CG)], wa0)
            cpb0.wait()
            swb0 = pltpu.async_copy(rb0, ob.at[pl.ds(c0, CG)], wb0)
            cpa1.wait()
            swa1 = pltpu.async_copy(ra1, oa.at[pl.ds(c1, CG)], wa1)
            cpb1.wait()
            swb1 = pltpu.async_copy(rb1, ob.at[pl.ds(c1, CG)], wb1)
            swa0.wait()
            swb0.wait()
            swa1.wait()
            swb1.wait()
            return carry

        lax.fori_loop(0, nch // 2, body, 0)

        if nch % 2:
            ct = pl.multiple_of(base + (nch - 1) * CG, 8)
            pltpu.sync_copy(ia.at[pl.ds(ct, CG)], iva0)
            pltpu.sync_copy(ib.at[pl.ds(ct, CG)], ivb0)
            cpa = pltpu.async_copy(tab.at[iva0], ra0, ga0)
            cpb = pltpu.async_copy(tab.at[ivb0], rb0, gb0)
            cpa.wait()
            swa = pltpu.async_copy(ra0, oa.at[pl.ds(ct, CG)], wa0)
            cpb.wait()
            swb = pltpu.async_copy(rb0, ob.at[pl.ds(ct, CG)], wb0)
            swa.wait()
            swb.wait()

    return gather2


def _make_scatter(E, H):
    perw = E // NWK
    nch = perw // CS
    rows_per_tile = NP // NS
    mesh = plsc.VectorSubcoreMesh(core_axis_name="c", subcore_axis_name="s")

    @functools.partial(
        pl.kernel, mesh=mesh,
        out_type=jax.ShapeDtypeStruct((NC, NP, H), f32),
        scratch_types=[pltpu.VMEM((CS,), jnp.int32),
                       pltpu.VMEM((CS, H), f32),
                       pltpu.SemaphoreType.DMA, pltpu.SemaphoreType.DMA,
                       pltpu.SemaphoreType.DMA,
                       pltpu.VMEM_SHARED((NP, H), f32)],
    )
    def scat(mref, cref, zref, out, iv, rv, li, lr, sa, acc):
        c = lax.axis_index("c")
        s = lax.axis_index("s")
        r0 = s * rows_per_tile
        pltpu.sync_copy(zref.at[pl.ds(r0, rows_per_tile)],
                        acc.at[pl.ds(r0, rows_per_tile)])
        plsc.subcore_barrier()
        base = (s * NC + c) * perw

        def body(i, carry):
            off = pl.multiple_of(base + i * CS, 8)
            hi = pltpu.async_copy(cref.at[pl.ds(off, CS)], iv, li)
            hr = pltpu.async_copy(mref.at[pl.ds(off, CS)], rv, lr)
            hi.wait()
            hr.wait()
            pltpu.async_copy(rv, acc.at[iv], sa, add=True).wait()
            return carry

        lax.fori_loop(0, nch, body, 0)
        plsc.subcore_barrier()
        pltpu.sync_copy(acc.at[pl.ds(r0, rows_per_tile)],
                        out.at[c, pl.ds(r0, rows_per_tile)])

    return scat


def kernel(x, edge_index, datanorm, W_in1, b_in1, W_in2, b_in2, W_in3,
           b_in3, W_c0a, b_c0a, W_c0b, b_c0b, W_c1a, b_c1a, W_c1b, b_c1b,
           W_e1, b_e1, W_e2, b_e2, W_e3, b_e3):
    N, D = x.shape
    E = edge_index.shape[1]
    H = W_in3.shape[1]
    F = H + D

    row = edge_index[0]
    col = edge_index[1]

    def pad2(w, r, c):
        return jnp.zeros((r, c), f32).at[:w.shape[0], :w.shape[1]].set(w)

    xp = jnp.zeros((NP, H), f32).at[:N, :D].set(x)
    dn = jnp.zeros((1, H), f32).at[0, :D].set(datanorm)
    zeros_nh = jnp.zeros((NP, H), f32)

    W1p = pad2(W_in1, 8, 2 * H)
    b1p = b_in1[None]
    b2p = b_in2[None]
    b3p = b_in3[None]

    def conv_prep(Wa, ba, Wb, bb):
        A = Wa[:F] - Wa[F:]
        B = Wa[F:]
        return (pad2(A, 2 * H, MWP), pad2(B, 2 * H, MWP),
                pad2(ba[None], 1, MWP), pad2(Wb, MWP, H), bb[None])

    AA0, BB0, ba0, Wb0, bb0 = conv_prep(W_c0a, b_c0a, W_c0b, b_c0b)
    AA1, BB1, ba1, Wb1, bb1 = conv_prep(W_c1a, b_c1a, W_c1b, b_c1b)

    W1a, W1c = W_e1[:2 * H], W_e1[2 * H:]
    be1 = b_e1[None]
    be2 = b_e2[None]
    W3p = pad2(W_e3, 2 * H, 8)
    b3f = jnp.full((1, 8), -1e30, f32).at[0, :W_e3.shape[1]].set(b_e3)

    E2 = E // 2
    gn = NP // BN
    ge = E2 // BE

    node0 = pl.pallas_call(
        _node0_body,
        grid=(gn,),
        in_specs=[_blk((BN, H)), _full((1, H)),
                  _full((8, 2 * H)), _full((1, 2 * H)),
                  _full((2 * H, 2 * H)), _full((1, 2 * H)),
                  _full((2 * H, H)), _full((1, H))],
        out_specs=_blk((BN, H)),
        out_shape=jax.ShapeDtypeStruct((NP, H), f32),
    )

    node1 = pl.pallas_call(
        _node1_body,
        grid=(gn,),
        in_specs=[_blk2((NC, BN, H)), _blk2((NC, BN, H)),
                  _blk((BN, H)), _full((1, H))],
        out_specs=[_blk((BN, H)), _blk((BN, H))],
        out_shape=[jax.ShapeDtypeStruct((NP, H), f32),
                   jax.ShapeDtypeStruct((NP, H), f32)],
    )

    node2 = pl.pallas_call(
        _node2_body,
        grid=(gn,),
        in_specs=[_blk2((NC, BN, H)), _blk2((NC, BN, H)), _blk((BN, H))],
        out_specs=_blk((BN, H)),
        out_shape=jax.ShapeDtypeStruct((NP, H), f32),
    )

    econv = pl.pallas_call(
        _econv_body,
        grid=(ge,),
        in_specs=[_blk((BE, H)), _blk((BE, H)),
                  _full((2 * H, MWP)), _full((2 * H, MWP)), _full((1, MWP)),
                  _full((MWP, H)), _full((1, H))],
        out_specs=_blk((BE, H)),
        out_shape=jax.ShapeDtypeStruct((E, H), f32),
    )

    efin = pl.pallas_call(
        _efin_body,
        grid=(ge,),
        in_specs=[_blk((BE, H)), _blk((BE, H)),
                  _full((2 * H, 2 * H)), _full((2 * H, 2 * H)),
                  _full((1, 2 * H)),
                  _full((2 * H, 2 * H)), _full((1, 2 * H)),
                  _full((2 * H, 8)), _full((1, 8))],
        out_specs=_blk((BE, 8)),
        out_shape=jax.ShapeDtypeStruct((E, 8), f32),
    )

    gather2 = _make_gather2(E2, H)
    scatter = _make_scatter(E2, H)

    cola, colb = col[:E2], col[E2:]
    rowa, rowb = row[:E2], row[E2:]

    def conv_phase(T, AA, BB, ba, Wb, bb):
        Gca, Gra = gather2(T, cola, rowa)
        Gcb, Grb = gather2(T, colb, rowb)
        ma = econv(Gca, Gra, AA, BB, ba, Wb, bb)
        mb = econv(Gcb, Grb, AA, BB, ba, Wb, bb)
        Hpa = scatter(ma, cola, zeros_nh)
        Hpb = scatter(mb, colb, zeros_nh)
        return Hpa, Hpb

    T0 = node0(xp, dn, W1p, b1p, W_in2, b2p, W_in3, b3p)
    Hp0a, Hp0b = conv_phase(T0, AA0, BB0, ba0, Wb0, bb0)

    H1, T1 = node1(Hp0a, Hp0b, xp, dn)
    Hp1a, Hp1b = conv_phase(T1, AA1, BB1, ba1, Wb1, bb1)

    T2 = node2(Hp1a, Hp1b, H1)
    Gr2a, Gc2a = gather2(T2, rowa, cola)
    Gr2b, Gc2b = gather2(T2, rowb, colb)
    o8a = efin(Gr2a, Gc2a, W1a, W1c, be1, W_e2, be2, W3p, b3f)
    o8b = efin(Gr2b, Gc2b, W1a, W1c, be1, W_e2, be2, W3p, b3f)
    return jnp.concatenate([o8a[:, :4], o8b[:, :4]], axis=0)


# speedup vs baseline: 4.3849x; 1.0466x over previous
import functools

import jax
import jax.numpy as jnp
from jax import lax
from jax.experimental import pallas as pl
from jax.experimental.pallas import tpu as pltpu
from jax.experimental.pallas import tpu_sc as plsc

f32 = jnp.float32
u32 = jnp.uint32

NP = 10240
MWP = 208
BN = 2048
BE = 3200
NC, NS = 2, 16
NWK = NC * NS
CG = 200
CS = 200


def _elu(v):
    return jnp.where(v > 0, v, jnp.exp(v) - 1.0)


def _dotp(a, b):
    return jax.lax.dot_general(a, b, (((1,), (0,)), ((), ())),
                               precision=lax.Precision.HIGHEST,
                               preferred_element_type=f32)


def _dot(a, b):
    return jax.lax.dot_general(a, b, (((1,), (0,)), ((), ())),
                               precision=lax.Precision.DEFAULT,
                               preferred_element_type=f32)


def _bits(v):
    return jax.lax.bitcast_convert_type(v, u32)


def _fl(w):
    return jax.lax.bitcast_convert_type(w, f32)


def _rne_hi(v):
    w = _bits(v)
    w = w + jnp.uint32(0x7FFF) + ((w >> 16) & jnp.uint32(1))
    return w & jnp.uint32(0xFFFF0000)


def _pack(hi, lo):
    return _fl(_rne_hi(hi) | (_rne_hi(lo) >> 16))


def _unpack(packed):
    w = _bits(packed)
    return jnp.concatenate(
        [_fl(w & jnp.uint32(0xFFFF0000)), _fl(w << 16)], axis=1)


def _node0_body(xr, dnr, w1r, b1r, w2r, b2r, w3r, b3r, t_out):
    xn = xr[...] * dnr[...]
    h = jnp.tanh(_dotp(xn[:, :8], w1r[...]) + b1r[...])
    h = jnp.tanh(_dotp(h, w2r[...]) + b2r[...])
    h0 = jnp.tanh(_dotp(h, w3r[...]) + b3r[...])
    t_out[...] = _pack(h0, xn)


def _node1_body(hpa, hpb, xr, dnr, h1_out, t_out):
    h1 = (hpa[0] + hpa[1]) + (hpb[0] + hpb[1])
    xn = xr[...] * dnr[...]
    h1_out[...] = h1
    t_out[...] = _pack(h1, xn)


def _node2_body(hpa, hpb, h1r, t_out):
    h2 = (hpa[0] + hpa[1]) + (hpb[0] + hpb[1])
    t_out[...] = _pack(h2, h1r[...])


def _econv_body(gcr, grr, aar, bbr2, bar, wbr, bbr, m_out):
    fc = _unpack(gcr[...])
    fr = _unpack(grr[...])
    m1 = _dot(fc, aar[...]) + _dot(fr, bbr2[...]) + bar[...]
    u = _elu(m1)
    m_out[...] = _elu(_dot(u, wbr[...]) + bbr[...])


def _efin_body(grr, gcr, w1a, w1c, b1r, w2r, b2r, w3r, b3r, o_out):
    fr = _unpack(grr[...])
    fc = _unpack(gcr[...])
    e1 = _elu(_dot(fr, w1a[...]) + _dot(fc, w1c[...]) + b1r[...])
    e2 = _elu(_dot(e1, w2r[...]) + b2r[...])
    lg = _dot(e2, w3r[...]) + b3r[...]
    mx = jnp.max(lg, axis=-1, keepdims=True)
    sh = lg - mx
    lse = jnp.log(jnp.sum(jnp.exp(sh), axis=-1, keepdims=True))
    o_out[...] = sh - lse


def _full(shape):
    return pl.BlockSpec(shape, lambda i: (0,) * len(shape))


def _blk(shape):
    def im(i):
        return (i,) + (0,) * (len(shape) - 1)
    return pl.BlockSpec(shape, im)


def _blk2(shape):
    def im(i):
        return (0, i) + (0,) * (len(shape) - 2)
    return pl.BlockSpec(shape, im)


def _make_gather2(E, W):
    perw = E // NWK
    nch = perw // CG
    mesh = plsc.VectorSubcoreMesh(core_axis_name="c", subcore_axis_name="s")

    @functools.partial(
        pl.kernel, mesh=mesh,
        out_type=[jax.ShapeDtypeStruct((E, W), f32),
                  jax.ShapeDtypeStruct((E, W), f32)],
        scratch_types=[pltpu.VMEM((CG,), jnp.int32),
                       pltpu.VMEM((CG,), jnp.int32),
                       pltpu.VMEM((CG,), jnp.int32),
                       pltpu.VMEM((CG,), jnp.int32),
                       pltpu.VMEM((CG, W), f32),
                       pltpu.VMEM((CG, W), f32),
                       pltpu.VMEM((CG, W), f32),
                       pltpu.VMEM((CG, W), f32),
                       pltpu.SemaphoreType.DMA, pltpu.SemaphoreType.DMA,
                       pltpu.SemaphoreType.DMA, pltpu.SemaphoreType.DMA,
                       pltpu.SemaphoreType.DMA, pltpu.SemaphoreType.DMA,
                       pltpu.SemaphoreType.DMA, pltpu.SemaphoreType.DMA],
    )
    def gather2(tab, ia, ib, oa, ob, iva0, ivb0, iva1, ivb1,
                ra0, rb0, ra1, rb1, ga0, gb0, ga1, gb1,
                wa0, wb0, wa1, wb1):
        wid = lax.axis_index("s") * NC + lax.axis_index("c")
        base = wid * perw

        def body(p, carry):
            c0 = pl.multiple_of(base + (2 * p) * CG, 8)
            c1 = pl.multiple_of(base + (2 * p + 1) * CG, 8)
            pltpu.sync_copy(ia.at[pl.ds(c0, CG)], iva0)
            pltpu.sync_copy(ib.at[pl.ds(c0, CG)], ivb0)
            cpa0 = pltpu.async_copy(tab.at[iva0], ra0, ga0)
            cpb0 = pltpu.async_copy(tab.at[ivb0], rb0, gb0)
            pltpu.sync_copy(ia.at[pl.ds(c1, CG)], iva1)
            pltpu.sync_copy(ib.at[pl.ds(c1, CG)], ivb1)
            cpa1 = pltpu.async_copy(tab.at[iva1], ra1, ga1)
            cpb1 = pltpu.async_copy(tab.at[ivb1], rb1, gb1)
            cpa0.wait()
            swa0 = pltpu.async_copy(ra0, oa.at[pl.ds(c0, CG)], wa0)
            cpb0.wait()
            swb0 = pltpu.async_copy(rb0, ob.at[pl.ds(c0, CG)], wb0)
            cpa1.wait()
            swa1 = pltpu.async_copy(ra1, oa.at[pl.ds(c1, CG)], wa1)
            cpb1.wait()
            swb1 = pltpu.async_copy(rb1, ob.at[pl.ds(c1, CG)], wb1)
            swa0.wait()
            swb0.wait()
            swa1.wait()
            swb1.wait()
            return carry

        lax.fori_loop(0, nch // 2, body, 0)

        if nch % 2:
            ct = pl.multiple_of(base + (nch - 1) * CG, 8)
            pltpu.sync_copy(ia.at[pl.ds(ct, CG)], iva0)
            pltpu.sync_copy(ib.at[pl.ds(ct, CG)], ivb0)
            cpa = pltpu.async_copy(tab.at[iva0], ra0, ga0)
            cpb = pltpu.async_copy(tab.at[ivb0], rb0, gb0)
            cpa.wait()
            swa = pltpu.async_copy(ra0, oa.at[pl.ds(ct, CG)], wa0)
            cpb.wait()
            swb = pltpu.async_copy(rb0, ob.at[pl.ds(ct, CG)], wb0)
            swa.wait()
            swb.wait()

    return gather2


def _make_scatter(E, H):
    perw = E // NWK
    nch = perw // CS
    rows_per_tile = NP // NS
    mesh = plsc.VectorSubcoreMesh(core_axis_name="c", subcore_axis_name="s")

    @functools.partial(
        pl.kernel, mesh=mesh,
        out_type=jax.ShapeDtypeStruct((NC, NP, H), f32),
        scratch_types=[pltpu.VMEM((CS,), jnp.int32),
                       pltpu.VMEM((CS, H), f32),
                       pltpu.SemaphoreType.DMA, pltpu.SemaphoreType.DMA,
                       pltpu.SemaphoreType.DMA,
                       pltpu.VMEM_SHARED((NP, H), f32)],
    )
    def scat(mref, cref, zref, out, iv, rv, li, lr, sa, acc):
        c = lax.axis_index("c")
        s = lax.axis_index("s")
        r0 = s * rows_per_tile
        pltpu.sync_copy(zref.at[pl.ds(r0, rows_per_tile)],
                        acc.at[pl.ds(r0, rows_per_tile)])
        plsc.subcore_barrier()
        base = (s * NC + c) * perw

        def body(i, carry):
            off = pl.multiple_of(base + i * CS, 8)
            hi = pltpu.async_copy(cref.at[pl.ds(off, CS)], iv, li)
            hr = pltpu.async_copy(mref.at[pl.ds(off, CS)], rv, lr)
            hi.wait()
            hr.wait()
            pltpu.async_copy(rv, acc.at[iv], sa, add=True).wait()
            return carry

        lax.fori_loop(0, nch, body, 0)
        plsc.subcore_barrier()
        pltpu.sync_copy(acc.at[pl.ds(r0, rows_per_tile)],
                        out.at[c, pl.ds(r0, rows_per_tile)])

    return scat


def kernel(x, edge_index, datanorm, W_in1, b_in1, W_in2, b_in2, W_in3,
           b_in3, W_c0a, b_c0a, W_c0b, b_c0b, W_c1a, b_c1a, W_c1b, b_c1b,
           W_e1, b_e1, W_e2, b_e2, W_e3, b_e3):
    N, D = x.shape
    E = edge_index.shape[1]
    H = W_in3.shape[1]
    F = H + D

    row = edge_index[0]
    col = edge_index[1]

    def pad2(w, r, c):
        return jnp.zeros((r, c), f32).at[:w.shape[0], :w.shape[1]].set(w)

    xp = jnp.zeros((NP, H), f32).at[:N, :D].set(x)
    dn = jnp.zeros((1, H), f32).at[0, :D].set(datanorm)
    zeros_nh = jnp.zeros((NP, H), f32)

    W1p = pad2(W_in1, 8, 2 * H)
    b1p = b_in1[None]
    b2p = b_in2[None]
    b3p = b_in3[None]

    def conv_prep(Wa, ba, Wb, bb):
        A = Wa[:F] - Wa[F:]
        B = Wa[F:]
        return (pad2(A, 2 * H, MWP), pad2(B, 2 * H, MWP),
                pad2(ba[None], 1, MWP), pad2(Wb, MWP, H), bb[None])

    AA0, BB0, ba0, Wb0, bb0 = conv_prep(W_c0a, b_c0a, W_c0b, b_c0b)
    AA1, BB1, ba1, Wb1, bb1 = conv_prep(W_c1a, b_c1a, W_c1b, b_c1b)

    W1a, W1c = W_e1[:2 * H], W_e1[2 * H:]
    be1 = b_e1[None]
    be2 = b_e2[None]
    W3p = pad2(W_e3, 2 * H, 8)
    b3f = jnp.full((1, 8), -1e30, f32).at[0, :W_e3.shape[1]].set(b_e3)

    E2 = E // 2
    gn = NP // BN
    ge = E2 // BE

    node0 = pl.pallas_call(
        _node0_body,
        grid=(gn,),
        in_specs=[_blk((BN, H)), _full((1, H)),
                  _full((8, 2 * H)), _full((1, 2 * H)),
                  _full((2 * H, 2 * H)), _full((1, 2 * H)),
                  _full((2 * H, H)), _full((1, H))],
        out_specs=_blk((BN, H)),
        out_shape=jax.ShapeDtypeStruct((NP, H), f32),
    )

    node1 = pl.pallas_call(
        _node1_body,
        grid=(gn,),
        in_specs=[_blk2((NC, BN, H)), _blk2((NC, BN, H)),
                  _blk((BN, H)), _full((1, H))],
        out_specs=[_blk((BN, H)), _blk((BN, H))],
        out_shape=[jax.ShapeDtypeStruct((NP, H), f32),
                   jax.ShapeDtypeStruct((NP, H), f32)],
    )

    node2 = pl.pallas_call(
        _node2_body,
        grid=(gn,),
        in_specs=[_blk2((NC, BN, H)), _blk2((NC, BN, H)), _blk((BN, H))],
        out_specs=_blk((BN, H)),
        out_shape=jax.ShapeDtypeStruct((NP, H), f32),
    )

    econv = pl.pallas_call(
        _econv_body,
        grid=(ge,),
        in_specs=[_blk((BE, H)), _blk((BE, H)),
                  _full((2 * H, MWP)), _full((2 * H, MWP)), _full((1, MWP)),
                  _full((MWP, H)), _full((1, H))],
        out_specs=_blk((BE, H)),
        out_shape=jax.ShapeDtypeStruct((E2, H), f32),
    )

    efin = pl.pallas_call(
        _efin_body,
        grid=(ge,),
        in_specs=[_blk((BE, H)), _blk((BE, H)),
                  _full((2 * H, 2 * H)), _full((2 * H, 2 * H)),
                  _full((1, 2 * H)),
                  _full((2 * H, 2 * H)), _full((1, 2 * H)),
                  _full((2 * H, 8)), _full((1, 8))],
        out_specs=_blk((BE, 8)),
        out_shape=jax.ShapeDtypeStruct((E2, 8), f32),
    )

    gather2 = _make_gather2(E2, H)
    scatter = _make_scatter(E2, H)

    cola, colb = col[:E2], col[E2:]
    rowa, rowb = row[:E2], row[E2:]

    def conv_phase(T, AA, BB, ba, Wb, bb):
        Gca, Gra = gather2(T, cola, rowa)
        Gcb, Grb = gather2(T, colb, rowb)
        ma = econv(Gca, Gra, AA, BB, ba, Wb, bb)
        mb = econv(Gcb, Grb, AA, BB, ba, Wb, bb)
        Hpa = scatter(ma, cola, zeros_nh)
        Hpb = scatter(mb, colb, zeros_nh)
        return Hpa, Hpb

    T0 = node0(xp, dn, W1p, b1p, W_in2, b2p, W_in3, b3p)
    Hp0a, Hp0b = conv_phase(T0, AA0, BB0, ba0, Wb0, bb0)

    H1, T1 = node1(Hp0a, Hp0b, xp, dn)
    Hp1a, Hp1b = conv_phase(T1, AA1, BB1, ba1, Wb1, bb1)

    T2 = node2(Hp1a, Hp1b, H1)
    Gr2a, Gc2a = gather2(T2, rowa, cola)
    Gr2b, Gc2b = gather2(T2, rowb, colb)
    o8a = efin(Gr2a, Gc2a, W1a, W1c, be1, W_e2, be2, W3p, b3f)
    o8b = efin(Gr2b, Gc2b, W1a, W1c, be1, W_e2, be2, W3p, b3f)
    return jnp.concatenate([o8a[:, :4], o8b[:, :4]], axis=0)


# speedup vs baseline: 4.9164x; 1.1212x over previous
import functools

import jax
import jax.numpy as jnp
from jax import lax
from jax.experimental import pallas as pl
from jax.experimental.pallas import tpu as pltpu
from jax.experimental.pallas import tpu_sc as plsc

f32 = jnp.float32
u32 = jnp.uint32

NP = 10240
MWP = 208
BN = 2048
BE = 3200
NC, NS = 2, 16
NWK = NC * NS
CG = 200
CS = 200


def _elu(v):
    return jnp.where(v > 0, v, jnp.exp(v) - 1.0)


def _dotp(a, b):
    return jax.lax.dot_general(a, b, (((1,), (0,)), ((), ())),
                               precision=lax.Precision.HIGHEST,
                               preferred_element_type=f32)


def _dot(a, b):
    return jax.lax.dot_general(a, b, (((1,), (0,)), ((), ())),
                               precision=lax.Precision.DEFAULT,
                               preferred_element_type=f32)


def _bits(v):
    return jax.lax.bitcast_convert_type(v, u32)


def _fl(w):
    return jax.lax.bitcast_convert_type(w, f32)


def _rne_hi(v):
    w = _bits(v)
    w = w + jnp.uint32(0x7FFF) + ((w >> 16) & jnp.uint32(1))
    return w & jnp.uint32(0xFFFF0000)


def _pack(hi, lo):
    return _fl(_rne_hi(hi) | (_rne_hi(lo) >> 16))


def _unpack(packed):
    w = _bits(packed)
    return jnp.concatenate(
        [_fl(w & jnp.uint32(0xFFFF0000)), _fl(w << 16)], axis=1)


def _node0_body(xr, dnr, w1r, b1r, w2r, b2r, w3r, b3r, t_out):
    xn = xr[...] * dnr[...]
    h = jnp.tanh(_dotp(xn[:, :8], w1r[...]) + b1r[...])
    h = jnp.tanh(_dotp(h, w2r[...]) + b2r[...])
    h0 = jnp.tanh(_dotp(h, w3r[...]) + b3r[...])
    t_out[...] = _pack(h0, xn)


def _node1_body(hpa, hpb, xr, dnr, h1_out, t_out):
    h1 = (hpa[0] + hpa[1]) + (hpb[0] + hpb[1])
    xn = xr[...] * dnr[...]
    h1_out[...] = h1
    t_out[...] = _pack(h1, xn)


def _node2_body(hpa, hpb, h1r, t_out):
    h2 = (hpa[0] + hpa[1]) + (hpb[0] + hpb[1])
    t_out[...] = _pack(h2, h1r[...])


def _econv_body(gcr, grr, aar, bbr2, bar, wbr, bbr, m_out):
    fc = _unpack(gcr[...])
    fr = _unpack(grr[...])
    m1 = _dot(fc, aar[...]) + _dot(fr, bbr2[...]) + bar[...]
    u = _elu(m1)
    m_out[...] = _elu(_dot(u, wbr[...]) + bbr[...])


def _efin_body(grr, gcr, w1a, w1c, b1r, w2r, b2r, w3r, b3r, o_out):
    fr = _unpack(grr[...])
    fc = _unpack(gcr[...])
    e1 = _elu(_dot(fr, w1a[...]) + _dot(fc, w1c[...]) + b1r[...])
    e2 = _elu(_dot(e1, w2r[...]) + b2r[...])
    lg = _dot(e2, w3r[...]) + b3r[...]
    mx = jnp.max(lg, axis=-1, keepdims=True)
    sh = lg - mx
    lse = jnp.log(jnp.sum(jnp.exp(sh), axis=-1, keepdims=True))
    o_out[...] = sh - lse


def _full(shape):
    return pl.BlockSpec(shape, lambda i: (0,) * len(shape))


def _blk(shape):
    def im(i):
        return (i,) + (0,) * (len(shape) - 1)
    return pl.BlockSpec(shape, im)


def _blk2(shape):
    def im(i):
        return (0, i) + (0,) * (len(shape) - 2)
    return pl.BlockSpec(shape, im)


def _make_gather2(E, W):
    perw = E // NWK
    CGV = 184
    NFULL = perw // CGV
    TAIL = perw - NFULL * CGV
    rows_per_tile = NP // NS
    mesh = plsc.VectorSubcoreMesh(core_axis_name="c", subcore_axis_name="s")

    @functools.partial(
        pl.kernel, mesh=mesh,
        out_type=[jax.ShapeDtypeStruct((E, W), f32),
                  jax.ShapeDtypeStruct((E, W), f32)],
        scratch_types=[pltpu.VMEM((CGV,), jnp.int32),
                       pltpu.VMEM((CGV,), jnp.int32),
                       pltpu.VMEM((TAIL,), jnp.int32),
                       pltpu.VMEM((TAIL,), jnp.int32),
                       pltpu.VMEM((CGV, W), f32),
                       pltpu.VMEM((CGV, W), f32),
                       pltpu.SemaphoreType.DMA, pltpu.SemaphoreType.DMA,
                       pltpu.SemaphoreType.DMA, pltpu.SemaphoreType.DMA,
                       pltpu.VMEM_SHARED((NP, W), f32)],
    )
    def gather2(tab, ia, ib, oa, ob, iva, ivb, ivta, ivtb,
                ra, rb, ga, gb, wa, wb, spm):
        s_ax = lax.axis_index("s")
        wid = s_ax * NC + lax.axis_index("c")
        r0 = s_ax * rows_per_tile
        pltpu.sync_copy(tab.at[pl.ds(r0, rows_per_tile)],
                        spm.at[pl.ds(r0, rows_per_tile)])
        plsc.subcore_barrier()
        base = wid * perw

        def pair(p, carry):
            c0 = pl.multiple_of(base + (2 * p) * CGV, 8)
            c1 = pl.multiple_of(base + (2 * p + 1) * CGV, 8)
            pltpu.sync_copy(ia.at[pl.ds(c0, CGV)], iva)
            pltpu.async_copy(spm.at[iva], ra, ga).wait()
            swa0 = pltpu.async_copy(ra, oa.at[pl.ds(c0, CGV)], wa)
            pltpu.sync_copy(ib.at[pl.ds(c0, CGV)], ivb)
            pltpu.async_copy(spm.at[ivb], rb, gb).wait()
            swb0 = pltpu.async_copy(rb, ob.at[pl.ds(c0, CGV)], wb)
            swa0.wait()
            pltpu.sync_copy(ia.at[pl.ds(c1, CGV)], iva)
            pltpu.async_copy(spm.at[iva], ra, ga).wait()
            swa1 = pltpu.async_copy(ra, oa.at[pl.ds(c1, CGV)], wa)
            swb0.wait()
            pltpu.sync_copy(ib.at[pl.ds(c1, CGV)], ivb)
            pltpu.async_copy(spm.at[ivb], rb, gb).wait()
            swb1 = pltpu.async_copy(rb, ob.at[pl.ds(c1, CGV)], wb)
            swa1.wait()
            swb1.wait()
            return carry

        lax.fori_loop(0, NFULL // 2, pair, 0)

        cl = pl.multiple_of(base + (NFULL - 1) * CGV, 8)
        pltpu.sync_copy(ia.at[pl.ds(cl, CGV)], iva)
        pltpu.async_copy(spm.at[iva], ra, ga).wait()
        swa = pltpu.async_copy(ra, oa.at[pl.ds(cl, CGV)], wa)
        pltpu.sync_copy(ib.at[pl.ds(cl, CGV)], ivb)
        pltpu.async_copy(spm.at[ivb], rb, gb).wait()
        swb = pltpu.async_copy(rb, ob.at[pl.ds(cl, CGV)], wb)

        ct = pl.multiple_of(base + NFULL * CGV, 8)
        pltpu.sync_copy(ia.at[pl.ds(ct, TAIL)], ivta)
        swa.wait()
        pltpu.async_copy(spm.at[ivta], ra.at[pl.ds(0, TAIL)], ga).wait()
        swa2 = pltpu.async_copy(ra.at[pl.ds(0, TAIL)],
                                oa.at[pl.ds(ct, TAIL)], wa)
        pltpu.sync_copy(ib.at[pl.ds(ct, TAIL)], ivtb)
        swb.wait()
        pltpu.async_copy(spm.at[ivtb], rb.at[pl.ds(0, TAIL)], gb).wait()
        swb2 = pltpu.async_copy(rb.at[pl.ds(0, TAIL)],
                                ob.at[pl.ds(ct, TAIL)], wb)
        swa2.wait()
        swb2.wait()

    return gather2


def _make_scatter(E, H):
    perw = E // NWK
    nch = perw // CS
    rows_per_tile = NP // NS
    mesh = plsc.VectorSubcoreMesh(core_axis_name="c", subcore_axis_name="s")

    @functools.partial(
        pl.kernel, mesh=mesh,
        out_type=jax.ShapeDtypeStruct((NC, NP, H), f32),
        scratch_types=[pltpu.VMEM((CS,), jnp.int32),
                       pltpu.VMEM((CS, H), f32),
                       pltpu.SemaphoreType.DMA, pltpu.SemaphoreType.DMA,
                       pltpu.SemaphoreType.DMA,
                       pltpu.VMEM_SHARED((NP, H), f32)],
    )
    def scat(mref, cref, zref, out, iv, rv, li, lr, sa, acc):
        c = lax.axis_index("c")
        s = lax.axis_index("s")
        r0 = s * rows_per_tile
        pltpu.sync_copy(zref.at[pl.ds(r0, rows_per_tile)],
                        acc.at[pl.ds(r0, rows_per_tile)])
        plsc.subcore_barrier()
        base = (s * NC + c) * perw

        def body(i, carry):
            off = pl.multiple_of(base + i * CS, 8)
            hi = pltpu.async_copy(cref.at[pl.ds(off, CS)], iv, li)
            hr = pltpu.async_copy(mref.at[pl.ds(off, CS)], rv, lr)
            hi.wait()
            hr.wait()
            pltpu.async_copy(rv, acc.at[iv], sa, add=True).wait()
            return carry

        lax.fori_loop(0, nch, body, 0)
        plsc.subcore_barrier()
        pltpu.sync_copy(acc.at[pl.ds(r0, rows_per_tile)],
                        out.at[c, pl.ds(r0, rows_per_tile)])

    return scat


def kernel(x, edge_index, datanorm, W_in1, b_in1, W_in2, b_in2, W_in3,
           b_in3, W_c0a, b_c0a, W_c0b, b_c0b, W_c1a, b_c1a, W_c1b, b_c1b,
           W_e1, b_e1, W_e2, b_e2, W_e3, b_e3):
    N, D = x.shape
    E = edge_index.shape[1]
    H = W_in3.shape[1]
    F = H + D

    row = edge_index[0]
    col = edge_index[1]

    def pad2(w, r, c):
        return jnp.zeros((r, c), f32).at[:w.shape[0], :w.shape[1]].set(w)

    xp = jnp.zeros((NP, H), f32).at[:N, :D].set(x)
    dn = jnp.zeros((1, H), f32).at[0, :D].set(datanorm)
    zeros_nh = jnp.zeros((NP, H), f32)

    W1p = pad2(W_in1, 8, 2 * H)
    b1p = b_in1[None]
    b2p = b_in2[None]
    b3p = b_in3[None]

    def conv_prep(Wa, ba, Wb, bb):
        A = Wa[:F] - Wa[F:]
        B = Wa[F:]
        return (pad2(A, 2 * H, MWP), pad2(B, 2 * H, MWP),
                pad2(ba[None], 1, MWP), pad2(Wb, MWP, H), bb[None])

    AA0, BB0, ba0, Wb0, bb0 = conv_prep(W_c0a, b_c0a, W_c0b, b_c0b)
    AA1, BB1, ba1, Wb1, bb1 = conv_prep(W_c1a, b_c1a, W_c1b, b_c1b)

    W1a, W1c = W_e1[:2 * H], W_e1[2 * H:]
    be1 = b_e1[None]
    be2 = b_e2[None]
    W3p = pad2(W_e3, 2 * H, 8)
    b3f = jnp.full((1, 8), -1e30, f32).at[0, :W_e3.shape[1]].set(b_e3)

    E2 = E // 2
    gn = NP // BN
    ge = E2 // BE

    node0 = pl.pallas_call(
        _node0_body,
        grid=(gn,),
        in_specs=[_blk((BN, H)), _full((1, H)),
                  _full((8, 2 * H)), _full((1, 2 * H)),
                  _full((2 * H, 2 * H)), _full((1, 2 * H)),
                  _full((2 * H, H)), _full((1, H))],
        out_specs=_blk((BN, H)),
        out_shape=jax.ShapeDtypeStruct((NP, H), f32),
    )

    node1 = pl.pallas_call(
        _node1_body,
        grid=(gn,),
        in_specs=[_blk2((NC, BN, H)), _blk2((NC, BN, H)),
                  _blk((BN, H)), _full((1, H))],
        out_specs=[_blk((BN, H)), _blk((BN, H))],
        out_shape=[jax.ShapeDtypeStruct((NP, H), f32),
                   jax.ShapeDtypeStruct((NP, H), f32)],
    )

    node2 = pl.pallas_call(
        _node2_body,
        grid=(gn,),
        in_specs=[_blk2((NC, BN, H)), _blk2((NC, BN, H)), _blk((BN, H))],
        out_specs=_blk((BN, H)),
        out_shape=jax.ShapeDtypeStruct((NP, H), f32),
    )

    econv = pl.pallas_call(
        _econv_body,
        grid=(ge,),
        in_specs=[_blk((BE, H)), _blk((BE, H)),
                  _full((2 * H, MWP)), _full((2 * H, MWP)), _full((1, MWP)),
                  _full((MWP, H)), _full((1, H))],
        out_specs=_blk((BE, H)),
        out_shape=jax.ShapeDtypeStruct((E2, H), f32),
    )

    efin = pl.pallas_call(
        _efin_body,
        grid=(ge,),
        in_specs=[_blk((BE, H)), _blk((BE, H)),
                  _full((2 * H, 2 * H)), _full((2 * H, 2 * H)),
                  _full((1, 2 * H)),
                  _full((2 * H, 2 * H)), _full((1, 2 * H)),
                  _full((2 * H, 8)), _full((1, 8))],
        out_specs=_blk((BE, 8)),
        out_shape=jax.ShapeDtypeStruct((E2, 8), f32),
    )

    gather2 = _make_gather2(E2, H)
    scatter = _make_scatter(E2, H)

    cola, colb = col[:E2], col[E2:]
    rowa, rowb = row[:E2], row[E2:]

    def conv_phase(T, AA, BB, ba, Wb, bb):
        Gca, Gra = gather2(T, cola, rowa)
        Gcb, Grb = gather2(T, colb, rowb)
        ma = econv(Gca, Gra, AA, BB, ba, Wb, bb)
        mb = econv(Gcb, Grb, AA, BB, ba, Wb, bb)
        Hpa = scatter(ma, cola, zeros_nh)
        Hpb = scatter(mb, colb, zeros_nh)
        return Hpa, Hpb

    T0 = node0(xp, dn, W1p, b1p, W_in2, b2p, W_in3, b3p)
    Hp0a, Hp0b = conv_phase(T0, AA0, BB0, ba0, Wb0, bb0)

    H1, T1 = node1(Hp0a, Hp0b, xp, dn)
    Hp1a, Hp1b = conv_phase(T1, AA1, BB1, ba1, Wb1, bb1)

    T2 = node2(Hp1a, Hp1b, H1)
    Gr2a, Gc2a = gather2(T2, rowa, cola)
    Gr2b, Gc2b = gather2(T2, rowb, colb)
    o8a = efin(Gr2a, Gc2a, W1a, W1c, be1, W_e2, be2, W3p, b3f)
    o8b = efin(Gr2b, Gc2b, W1a, W1c, be1, W_e2, be2, W3p, b3f)
    return jnp.concatenate([o8a[:, :4], o8b[:, :4]], axis=0)


# speedup vs baseline: 4.9245x; 1.0016x over previous
import functools

import jax
import jax.numpy as jnp
from jax import lax
from jax.experimental import pallas as pl
from jax.experimental.pallas import tpu as pltpu
from jax.experimental.pallas import tpu_sc as plsc

f32 = jnp.float32
u32 = jnp.uint32

NP = 10240
MWP = 208
BN = 2048
BE = 3200
NC, NS = 2, 16
NWK = NC * NS
CG = 200
CS = 200


def _elu(v):
    return jnp.where(v > 0, v, jnp.exp(v) - 1.0)


def _dotp(a, b):
    return jax.lax.dot_general(a, b, (((1,), (0,)), ((), ())),
                               precision=lax.Precision.HIGHEST,
                               preferred_element_type=f32)


def _dot(a, b):
    return jax.lax.dot_general(a, b, (((1,), (0,)), ((), ())),
                               precision=lax.Precision.DEFAULT,
                               preferred_element_type=f32)


def _bits(v):
    return jax.lax.bitcast_convert_type(v, u32)


def _fl(w):
    return jax.lax.bitcast_convert_type(w, f32)


def _rne_hi(v):
    w = _bits(v)
    w = w + jnp.uint32(0x7FFF) + ((w >> 16) & jnp.uint32(1))
    return w & jnp.uint32(0xFFFF0000)


def _pack(hi, lo):
    return _fl(_rne_hi(hi) | (_rne_hi(lo) >> 16))


def _unpack(packed):
    w = _bits(packed)
    return jnp.concatenate(
        [_fl(w & jnp.uint32(0xFFFF0000)), _fl(w << 16)], axis=1)


def _node0_body(xr, dnr, w1r, b1r, w2r, b2r, w3r, b3r, t_out):
    xn = xr[...] * dnr[...]
    h = jnp.tanh(_dotp(xn[:, :8], w1r[...]) + b1r[...])
    h = jnp.tanh(_dotp(h, w2r[...]) + b2r[...])
    h0 = jnp.tanh(_dotp(h, w3r[...]) + b3r[...])
    t_out[...] = _pack(h0, xn)


def _node1_body(hpa, hpb, xr, dnr, h1_out, t_out):
    h1 = (hpa[0] + hpa[1]) + (hpb[0] + hpb[1])
    xn = xr[...] * dnr[...]
    h1_out[...] = h1
    t_out[...] = _pack(h1, xn)


def _node2_body(hpa, hpb, h1r, t_out):
    h2 = (hpa[0] + hpa[1]) + (hpb[0] + hpb[1])
    t_out[...] = _pack(h2, h1r[...])


def _econv_body(gcr, grr, aar, bbr2, bar, wbr, bbr, m_out):
    fc = _unpack(gcr[...])
    fr = _unpack(grr[...])
    m1 = _dot(fc, aar[...]) + _dot(fr, bbr2[...]) + bar[...]
    u = _elu(m1)
    m_out[...] = _elu(_dot(u, wbr[...]) + bbr[...])


def _efin_body(grr, gcr, w1a, w1c, b1r, w2r, b2r, w3r, b3r, o_out):
    fr = _unpack(grr[...])
    fc = _unpack(gcr[...])
    e1 = _elu(_dot(fr, w1a[...]) + _dot(fc, w1c[...]) + b1r[...])
    e2 = _elu(_dot(e1, w2r[...]) + b2r[...])
    lg = _dot(e2, w3r[...]) + b3r[...]
    mx = jnp.max(lg, axis=-1, keepdims=True)
    sh = lg - mx
    lse = jnp.log(jnp.sum(jnp.exp(sh), axis=-1, keepdims=True))
    o_out[...] = sh - lse


def _full(shape):
    return pl.BlockSpec(shape, lambda i: (0,) * len(shape))


def _blk(shape):
    def im(i):
        return (i,) + (0,) * (len(shape) - 1)
    return pl.BlockSpec(shape, im)


def _blk2(shape):
    def im(i):
        return (0, i) + (0,) * (len(shape) - 2)
    return pl.BlockSpec(shape, im)


def _make_gather2(E, W):
    perw = E // NWK
    CGV = 184
    NFULL = perw // CGV
    TAIL = perw - NFULL * CGV
    rows_per_tile = NP // NS
    mesh = plsc.VectorSubcoreMesh(core_axis_name="c", subcore_axis_name="s")

    @functools.partial(
        pl.kernel, mesh=mesh,
        out_type=[jax.ShapeDtypeStruct((E, W), f32),
                  jax.ShapeDtypeStruct((E, W), f32)],
        scratch_types=[pltpu.VMEM((CGV,), jnp.int32),
                       pltpu.VMEM((CGV,), jnp.int32),
                       pltpu.VMEM((TAIL,), jnp.int32),
                       pltpu.VMEM((TAIL,), jnp.int32),
                       pltpu.VMEM((CGV, W), f32),
                       pltpu.VMEM((CGV, W), f32),
                       pltpu.SemaphoreType.DMA, pltpu.SemaphoreType.DMA,
                       pltpu.SemaphoreType.DMA, pltpu.SemaphoreType.DMA,
                       pltpu.VMEM_SHARED((NP, W), f32)],
    )
    def gather2(tab, ia, ib, oa, ob, iva, ivb, ivta, ivtb,
                ra, rb, ga, gb, wa, wb, spm):
        s_ax = lax.axis_index("s")
        wid = s_ax * NC + lax.axis_index("c")
        r0 = s_ax * rows_per_tile
        pltpu.sync_copy(tab.at[pl.ds(r0, rows_per_tile)],
                        spm.at[pl.ds(r0, rows_per_tile)])
        plsc.subcore_barrier()
        base = wid * perw

        def pair(p, carry):
            c0 = pl.multiple_of(base + (2 * p) * CGV, 8)
            c1 = pl.multiple_of(base + (2 * p + 1) * CGV, 8)
            pltpu.sync_copy(ia.at[pl.ds(c0, CGV)], iva)
            pltpu.async_copy(spm.at[iva], ra, ga).wait()
            swa0 = pltpu.async_copy(ra, oa.at[pl.ds(c0, CGV)], wa)
            pltpu.sync_copy(ib.at[pl.ds(c0, CGV)], ivb)
            pltpu.async_copy(spm.at[ivb], rb, gb).wait()
            swb0 = pltpu.async_copy(rb, ob.at[pl.ds(c0, CGV)], wb)
            swa0.wait()
            pltpu.sync_copy(ia.at[pl.ds(c1, CGV)], iva)
            pltpu.async_copy(spm.at[iva], ra, ga).wait()
            swa1 = pltpu.async_copy(ra, oa.at[pl.ds(c1, CGV)], wa)
            swb0.wait()
            pltpu.sync_copy(ib.at[pl.ds(c1, CGV)], ivb)
            pltpu.async_copy(spm.at[ivb], rb, gb).wait()
            swb1 = pltpu.async_copy(rb, ob.at[pl.ds(c1, CGV)], wb)
            swa1.wait()
            swb1.wait()
            return carry

        lax.fori_loop(0, NFULL // 2, pair, 0)

        cl = pl.multiple_of(base + (NFULL - 1) * CGV, 8)
        pltpu.sync_copy(ia.at[pl.ds(cl, CGV)], iva)
        pltpu.async_copy(spm.at[iva], ra, ga).wait()
        swa = pltpu.async_copy(ra, oa.at[pl.ds(cl, CGV)], wa)
        pltpu.sync_copy(ib.at[pl.ds(cl, CGV)], ivb)
        pltpu.async_copy(spm.at[ivb], rb, gb).wait()
        swb = pltpu.async_copy(rb, ob.at[pl.ds(cl, CGV)], wb)

        ct = pl.multiple_of(base + NFULL * CGV, 8)
        pltpu.sync_copy(ia.at[pl.ds(ct, TAIL)], ivta)
        swa.wait()
        pltpu.async_copy(spm.at[ivta], ra.at[pl.ds(0, TAIL)], ga).wait()
        swa2 = pltpu.async_copy(ra.at[pl.ds(0, TAIL)],
                                oa.at[pl.ds(ct, TAIL)], wa)
        pltpu.sync_copy(ib.at[pl.ds(ct, TAIL)], ivtb)
        swb.wait()
        pltpu.async_copy(spm.at[ivtb], rb.at[pl.ds(0, TAIL)], gb).wait()
        swb2 = pltpu.async_copy(rb.at[pl.ds(0, TAIL)],
                                ob.at[pl.ds(ct, TAIL)], wb)
        swa2.wait()
        swb2.wait()

    return gather2


def _make_scatter(E, H):
    perw = E // NWK
    CSV = 184
    NFULL = perw // CSV
    TAIL = perw - NFULL * CSV
    rows_per_tile = NP // NS
    mesh = plsc.VectorSubcoreMesh(core_axis_name="c", subcore_axis_name="s")

    @functools.partial(
        pl.kernel, mesh=mesh,
        out_type=jax.ShapeDtypeStruct((NC, NP, H), f32),
        scratch_types=[pltpu.VMEM((CSV,), jnp.int32),
                       pltpu.VMEM((CSV,), jnp.int32),
                       pltpu.VMEM((TAIL,), jnp.int32),
                       pltpu.VMEM((CSV, H), f32),
                       pltpu.VMEM((CSV, H), f32),
                       pltpu.SemaphoreType.DMA, pltpu.SemaphoreType.DMA,
                       pltpu.SemaphoreType.DMA, pltpu.SemaphoreType.DMA,
                       pltpu.SemaphoreType.DMA, pltpu.SemaphoreType.DMA,
                       pltpu.VMEM_SHARED((NP, H), f32)],
    )
    def scat(mref, cref, zref, out, iv0, iv1, ivt, rv0, rv1,
             li0, li1, lr0, lr1, s0, s1, acc):
        c = lax.axis_index("c")
        s = lax.axis_index("s")
        r0 = s * rows_per_tile
        pltpu.sync_copy(zref.at[pl.ds(r0, rows_per_tile)],
                        acc.at[pl.ds(r0, rows_per_tile)])
        plsc.subcore_barrier()
        base = (s * NC + c) * perw

        def pair(p, carry):
            c0 = pl.multiple_of(base + (2 * p) * CSV, 8)
            c1 = pl.multiple_of(base + (2 * p + 1) * CSV, 8)
            hi0 = pltpu.async_copy(cref.at[pl.ds(c0, CSV)], iv0, li0)
            hr0 = pltpu.async_copy(mref.at[pl.ds(c0, CSV)], rv0, lr0)
            hi1 = pltpu.async_copy(cref.at[pl.ds(c1, CSV)], iv1, li1)
            hr1 = pltpu.async_copy(mref.at[pl.ds(c1, CSV)], rv1, lr1)
            hi0.wait()
            hr0.wait()
            sc0 = pltpu.async_copy(rv0, acc.at[iv0], s0, add=True)
            hi1.wait()
            hr1.wait()
            sc0.wait()
            pltpu.async_copy(rv1, acc.at[iv1], s1, add=True).wait()
            return carry

        lax.fori_loop(0, NFULL // 2, pair, 0)

        cl = pl.multiple_of(base + (NFULL - 1) * CSV, 8)
        hi0 = pltpu.async_copy(cref.at[pl.ds(cl, CSV)], iv0, li0)
        hr0 = pltpu.async_copy(mref.at[pl.ds(cl, CSV)], rv0, lr0)
        ct = pl.multiple_of(base + NFULL * CSV, 8)
        hi1 = pltpu.async_copy(cref.at[pl.ds(ct, TAIL)], ivt, li1)
        hr1 = pltpu.async_copy(mref.at[pl.ds(ct, TAIL)],
                               rv1.at[pl.ds(0, TAIL)], lr1)
        hi0.wait()
        hr0.wait()
        sc0 = pltpu.async_copy(rv0, acc.at[iv0], s0, add=True)
        hi1.wait()
        hr1.wait()
        sc0.wait()
        pltpu.async_copy(rv1.at[pl.ds(0, TAIL)], acc.at[ivt], s1,
                         add=True).wait()

        plsc.subcore_barrier()
        pltpu.sync_copy(acc.at[pl.ds(r0, rows_per_tile)],
                        out.at[c, pl.ds(r0, rows_per_tile)])

    return scat


def kernel(x, edge_index, datanorm, W_in1, b_in1, W_in2, b_in2, W_in3,
           b_in3, W_c0a, b_c0a, W_c0b, b_c0b, W_c1a, b_c1a, W_c1b, b_c1b,
           W_e1, b_e1, W_e2, b_e2, W_e3, b_e3):
    N, D = x.shape
    E = edge_index.shape[1]
    H = W_in3.shape[1]
    F = H + D

    row = edge_index[0]
    col = edge_index[1]

    def pad2(w, r, c):
        return jnp.zeros((r, c), f32).at[:w.shape[0], :w.shape[1]].set(w)

    xp = jnp.zeros((NP, H), f32).at[:N, :D].set(x)
    dn = jnp.zeros((1, H), f32).at[0, :D].set(datanorm)
    zeros_nh = jnp.zeros((NP, H), f32)

    W1p = pad2(W_in1, 8, 2 * H)
    b1p = b_in1[None]
    b2p = b_in2[None]
    b3p = b_in3[None]

    def conv_prep(Wa, ba, Wb, bb):
        A = Wa[:F] - Wa[F:]
        B = Wa[F:]
        return (pad2(A, 2 * H, MWP), pad2(B, 2 * H, MWP),
                pad2(ba[None], 1, MWP), pad2(Wb, MWP, H), bb[None])

    AA0, BB0, ba0, Wb0, bb0 = conv_prep(W_c0a, b_c0a, W_c0b, b_c0b)
    AA1, BB1, ba1, Wb1, bb1 = conv_prep(W_c1a, b_c1a, W_c1b, b_c1b)

    W1a, W1c = W_e1[:2 * H], W_e1[2 * H:]
    be1 = b_e1[None]
    be2 = b_e2[None]
    W3p = pad2(W_e3, 2 * H, 8)
    b3f = jnp.full((1, 8), -1e30, f32).at[0, :W_e3.shape[1]].set(b_e3)

    E2 = E // 2
    gn = NP // BN
    ge = E2 // BE

    node0 = pl.pallas_call(
        _node0_body,
        grid=(gn,),
        in_specs=[_blk((BN, H)), _full((1, H)),
                  _full((8, 2 * H)), _full((1, 2 * H)),
                  _full((2 * H, 2 * H)), _full((1, 2 * H)),
                  _full((2 * H, H)), _full((1, H))],
        out_specs=_blk((BN, H)),
        out_shape=jax.ShapeDtypeStruct((NP, H), f32),
    )

    node1 = pl.pallas_call(
        _node1_body,
        grid=(gn,),
        in_specs=[_blk2((NC, BN, H)), _blk2((NC, BN, H)),
                  _blk((BN, H)), _full((1, H))],
        out_specs=[_blk((BN, H)), _blk((BN, H))],
        out_shape=[jax.ShapeDtypeStruct((NP, H), f32),
                   jax.ShapeDtypeStruct((NP, H), f32)],
    )

    node2 = pl.pallas_call(
        _node2_body,
        grid=(gn,),
        in_specs=[_blk2((NC, BN, H)), _blk2((NC, BN, H)), _blk((BN, H))],
        out_specs=_blk((BN, H)),
        out_shape=jax.ShapeDtypeStruct((NP, H), f32),
    )

    econv = pl.pallas_call(
        _econv_body,
        grid=(ge,),
        in_specs=[_blk((BE, H)), _blk((BE, H)),
                  _full((2 * H, MWP)), _full((2 * H, MWP)), _full((1, MWP)),
                  _full((MWP, H)), _full((1, H))],
        out_specs=_blk((BE, H)),
        out_shape=jax.ShapeDtypeStruct((E2, H), f32),
    )

    efin = pl.pallas_call(
        _efin_body,
        grid=(ge,),
        in_specs=[_blk((BE, H)), _blk((BE, H)),
                  _full((2 * H, 2 * H)), _full((2 * H, 2 * H)),
                  _full((1, 2 * H)),
                  _full((2 * H, 2 * H)), _full((1, 2 * H)),
                  _full((2 * H, 8)), _full((1, 8))],
        out_specs=_blk((BE, 8)),
        out_shape=jax.ShapeDtypeStruct((E2, 8), f32),
    )

    gather2 = _make_gather2(E2, H)
    scatter = _make_scatter(E2, H)

    cola, colb = col[:E2], col[E2:]
    rowa, rowb = row[:E2], row[E2:]

    def conv_phase(T, AA, BB, ba, Wb, bb):
        Gca, Gra = gather2(T, cola, rowa)
        Gcb, Grb = gather2(T, colb, rowb)
        ma = econv(Gca, Gra, AA, BB, ba, Wb, bb)
        mb = econv(Gcb, Grb, AA, BB, ba, Wb, bb)
        Hpa = scatter(ma, cola, zeros_nh)
        Hpb = scatter(mb, colb, zeros_nh)
        return Hpa, Hpb

    T0 = node0(xp, dn, W1p, b1p, W_in2, b2p, W_in3, b3p)
    Hp0a, Hp0b = conv_phase(T0, AA0, BB0, ba0, Wb0, bb0)

    H1, T1 = node1(Hp0a, Hp0b, xp, dn)
    Hp1a, Hp1b = conv_phase(T1, AA1, BB1, ba1, Wb1, bb1)

    T2 = node2(Hp1a, Hp1b, H1)
    Gr2a, Gc2a = gather2(T2, rowa, cola)
    Gr2b, Gc2b = gather2(T2, rowb, colb)
    o8a = efin(Gr2a, Gc2a, W1a, W1c, be1, W_e2, be2, W3p, b3f)
    o8b = efin(Gr2b, Gc2b, W1a, W1c, be1, W_e2, be2, W3p, b3f)
    return jnp.concatenate([o8a[:, :4], o8b[:, :4]], axis=0)
